# baseline (speedup 1.0000x reference)
import functools

import jax
import jax.numpy as jnp
from jax import lax
from jax.experimental import pallas as pl
from jax.experimental.pallas import tpu as pltpu
from jax.experimental.pallas import tpu_sc as plsc

N_OBJ = 10000
N_PRED = 160000
H = 512
DOUT = 128

NC = 2
NS = 16
NW = NC * NS

E_PAD = 163840
NODE_PAD = 10016
A_ROWS = 10008

G_EPT = E_PAD // NW
G_CH = 40
G_NCH = G_EPT // G_CH

S_EPT = E_PAD // NS
S_CH = 40
S_NCH = S_EPT // S_CH
S_NPH = 2
S_PCH = S_NCH // S_NPH
S_K = 2
S_NV = S_PCH // S_K
SLAB = 128
RPT = 624
RPT_LAST = N_OBJ - (NS - 1) * RPT


def _sc_mesh():
    return plsc.VectorSubcoreMesh(core_axis_name="c", subcore_axis_name="s")


G_K = 4
G_HROWS = G_K * G_CH
G_NV = G_NCH // G_K


def _gather_body(d, table, sidx, oidx, gs_out, go_out, idx_v, stage, gsems,
                 osems):
    c = lax.axis_index("c")
    s = lax.axis_index("s")
    wid = s * NC + c
    base = wid * G_EPT

    def run(idx_hbm, out_hbm):
        pltpu.sync_copy(idx_hbm.at[wid], idx_v)

        def pair(dd, _):
            for h in range(2):
                v = 2 * dd + h

                @pl.when(v >= 2)
                def _():
                    pltpu.make_async_copy(
                        stage.at[h], out_hbm.at[pl.ds(base, G_HROWS)],
                        osems.at[h]).wait()

                for k in range(G_K):
                    pltpu.async_copy(
                        table.at[idx_v.at[v * G_K + k]],
                        stage.at[h, pl.ds(k * G_CH, G_CH)], gsems.at[h])
                for k in range(G_K):
                    pltpu.make_async_copy(
                        table.at[idx_v.at[0]],
                        stage.at[h, pl.ds(k * G_CH, G_CH)],
                        gsems.at[h]).wait()
                pltpu.async_copy(
                    stage.at[h],
                    out_hbm.at[pl.ds(base + v * G_HROWS, G_HROWS)],
                    osems.at[h])
            return ()

        lax.fori_loop(0, G_NV // 2, pair, (), unroll=False)
        for h in range(2):
            pltpu.make_async_copy(
                stage.at[h], out_hbm.at[pl.ds(base, G_HROWS)],
                osems.at[h]).wait()

    run(sidx, gs_out)
    run(oidx, go_out)


def _gather(table, sidx_g, oidx_g):
    d = table.shape[1]
    kfn = pl.kernel(
        functools.partial(_gather_body, d),
        out_type=(
            jax.ShapeDtypeStruct((E_PAD, d), jnp.float32),
            jax.ShapeDtypeStruct((E_PAD, d), jnp.float32),
        ),
        mesh=_sc_mesh(),
        scratch_types=[
            pltpu.VMEM((G_NCH, G_CH), jnp.int32),
            pltpu.VMEM((2, G_HROWS, d), jnp.float32),
            pltpu.SemaphoreType.DMA((2,)),
            pltpu.SemaphoreType.DMA((2,)),
        ],
    )
    return kfn(table, sidx_g, oidx_g)


def _fill(ref, rows, cols, value):
    v = jnp.full((16,), value, jnp.float32)

    def zrow(r, _):
        def zcol(k, _):
            ref[r, pl.ds(k * 16, 16)] = v
            return ()
        lax.fori_loop(0, cols // 16, zcol, (), unroll=True)
        return ()

    lax.fori_loop(0, rows, zrow, (), unroll=False)


def _own_rows(s, fn):
    @pl.when(s < NS - 1)
    def _():
        fn(s * RPT, RPT)

    @pl.when(s == NS - 1)
    def _():
        fn((NS - 1) * RPT, RPT_LAST)


def _zero_rows(acc, zbuf, s):
    def do(r0, n):
        def st(i, _):
            pltpu.sync_copy(zbuf, acc.at[pl.ds(r0 + i * 8, 8)])
            return ()
        lax.fori_loop(0, n // 8, st, (), unroll=False)
    _own_rows(s, do)


def _scatter_body(vs, vo, sidx, oidx, out, acc, idx_v, vals, zbuf,
                  rsems, ssems):
    c = lax.axis_index("c")
    s = lax.axis_index("s")
    _fill(zbuf, 8, SLAB, 0.0)

    for p in range(2):
        col0 = c * (2 * SLAB) + p * SLAB
        _zero_rows(acc, zbuf, s)
        plsc.subcore_barrier()
        for idx_hbm, val_hbm in ((sidx, vs), (oidx, vo)):
            for ph in range(S_NPH):
                pltpu.sync_copy(idx_hbm.at[s, pl.ds(ph * S_PCH, S_PCH)],
                                idx_v)
                ebase = s * S_EPT + ph * S_PCH * S_CH

                def pair(dd, _):
                    for h in range(2):
                        v = 2 * dd + h

                        @pl.when(v >= 2)
                        def _():
                            for k in range(S_K):
                                pltpu.make_async_copy(
                                    vals.at[h, pl.ds(k * S_CH, S_CH)],
                                    acc.at[pl.ds(0, S_CH)],
                                    ssems.at[h]).wait()

                        for k in range(S_K):
                            pltpu.async_copy(
                                val_hbm.at[
                                    pl.ds(ebase + (v * S_K + k) * S_CH,
                                          S_CH),
                                    pl.ds(col0, SLAB)],
                                vals.at[h, pl.ds(k * S_CH, S_CH)],
                                rsems.at[h])
                        for k in range(S_K):
                            pltpu.make_async_copy(
                                val_hbm.at[pl.ds(s * S_EPT, S_CH),
                                           pl.ds(col0, SLAB)],
                                vals.at[h, pl.ds(k * S_CH, S_CH)],
                                rsems.at[h]).wait()
                        for k in range(S_K):
                            pltpu.async_copy(
                                vals.at[h, pl.ds(k * S_CH, S_CH)],
                                acc.at[idx_v.at[v * S_K + k]], ssems.at[h],
                                add=True)
                    return ()

                lax.fori_loop(0, S_NV // 2, pair, (), unroll=False)
                for h in range(2):
                    for k in range(S_K):
                        pltpu.make_async_copy(
                            vals.at[h, pl.ds(k * S_CH, S_CH)],
                            acc.at[pl.ds(0, S_CH)], ssems.at[h]).wait()
        plsc.subcore_barrier()
        _own_rows(s, lambda r0, n: pltpu.sync_copy(
            acc.at[pl.ds(r0, n)],
            out.at[pl.ds(r0, n), pl.ds(col0, SLAB)]))


def _scatter(vs, vo, sidx_s, oidx_s):
    kfn = pl.kernel(
        _scatter_body,
        out_type=jax.ShapeDtypeStruct((N_OBJ, H), jnp.float32),
        mesh=_sc_mesh(),
        scratch_types=[
            pltpu.VMEM_SHARED((A_ROWS, SLAB), jnp.float32),
            pltpu.VMEM((S_PCH, S_CH), jnp.int32),
            pltpu.VMEM((2, S_K * S_CH, SLAB), jnp.float32),
            pltpu.VMEM((8, SLAB), jnp.float32),
            pltpu.SemaphoreType.DMA((2,)),
            pltpu.SemaphoreType.DMA((2,)),
        ],
    )
    return kfn(vs, vo, sidx_s, oidx_s)


CW = 128


def _counts_body(sidx, oidx, out, acc, idx_v, ones_v, zeros_v):
    c = lax.axis_index("c")
    s = lax.axis_index("s")

    @pl.when(c == 0)
    def _():
        _fill(ones_v, S_CH, CW, 1.0)
        _fill(zeros_v, 8, CW, 0.0)
        _zero_rows(acc, zeros_v, s)
        plsc.subcore_barrier()
        for idx_hbm in (sidx, oidx):
            pltpu.sync_copy(idx_hbm.at[s], idx_v)

            def step(j, _):
                pltpu.sync_copy(ones_v, acc.at[idx_v.at[j]], add=True)
                return ()

            lax.fori_loop(0, S_NCH, step, (), unroll=False)
        plsc.subcore_barrier()
        _own_rows(s, lambda r0, n: pltpu.sync_copy(
            acc.at[pl.ds(r0, n)], out.at[pl.ds(r0, n)]))


def _counts(sidx_s, oidx_s):
    kfn = pl.kernel(
        _counts_body,
        out_type=jax.ShapeDtypeStruct((N_OBJ, CW), jnp.float32),
        mesh=_sc_mesh(),
        scratch_types=[
            pltpu.VMEM_SHARED((A_ROWS, CW), jnp.float32),
            pltpu.VMEM((S_NCH, S_CH), jnp.int32),
            pltpu.VMEM((S_CH, CW), jnp.float32),
            pltpu.VMEM((8, CW), jnp.float32),
        ],
    )
    return kfn(sidx_s, oidx_s)


BE = 1024
BN = 1000


def _edge_mlp_body(din, gs, pred, go, w1, b1, w2, b2, ns, np_, no):
    h = jnp.dot(gs[:, :din], w1[:din, :], preferred_element_type=jnp.float32)
    h += jnp.dot(pred[...], w1[din:2 * din, :],
                 preferred_element_type=jnp.float32)
    h += jnp.dot(go[:, :din], w1[2 * din:, :],
                 preferred_element_type=jnp.float32)
    h = jax.nn.relu(h + b1[...])
    ns[...] = jax.nn.relu(
        jnp.dot(h, w2[:, :H], preferred_element_type=jnp.float32)
        + b2[:, :H])
    np_[...] = jax.nn.relu(
        jnp.dot(h, w2[:, H:H + DOUT], preferred_element_type=jnp.float32)
        + b2[:, H:H + DOUT])
    no[...] = jax.nn.relu(
        jnp.dot(h, w2[:, H + DOUT:], preferred_element_type=jnp.float32)
        + b2[:, H + DOUT:])


def _edge_mlp(gs, pred, go, w1, b1, w2, b2):
    din = w1.shape[0] // 3
    dg = gs.shape[1]
    grid = (E_PAD // BE,)
    row = lambda i: (i, 0)
    full = lambda i: (0, 0)
    return pl.pallas_call(
        functools.partial(_edge_mlp_body, din),
        grid=grid,
        in_specs=[
            pl.BlockSpec((BE, dg), row),
            pl.BlockSpec((BE, din), row),
            pl.BlockSpec((BE, dg), row),
            pl.BlockSpec(w1.shape, full),
            pl.BlockSpec(b1.shape, full),
            pl.BlockSpec(w2.shape, full),
            pl.BlockSpec(b2.shape, full),
        ],
        out_specs=[
            pl.BlockSpec((BE, H), row),
            pl.BlockSpec((BE, DOUT), row),
            pl.BlockSpec((BE, H), row),
        ],
        out_shape=[
            jax.ShapeDtypeStruct((E_PAD, H), jnp.float32),
            jax.ShapeDtypeStruct((E_PAD, DOUT), jnp.float32),
            jax.ShapeDtypeStruct((E_PAD, H), jnp.float32),
        ],
    )(gs, pred, go, w1, b1, w2, b2)


def _node_mlp_body(pooled, cnt, w3, b3, w4, b4, out):
    c = cnt[:, 0:1]
    inv = 1.0 / jnp.maximum(c, 1.0)
    h2 = jax.nn.relu(
        jnp.dot(pooled[...] * inv, w3[...], preferred_element_type=jnp.float32)
        + b3[...])
    out[...] = jnp.dot(h2, w4[...], preferred_element_type=jnp.float32) + b4[...]


def _node_mlp(pooled, cnt, w3, b3, w4, b4):
    grid = (N_OBJ // BN,)
    row = lambda i: (i, 0)
    full = lambda i: (0, 0)
    return pl.pallas_call(
        _node_mlp_body,
        grid=grid,
        in_specs=[
            pl.BlockSpec((BN, H), row),
            pl.BlockSpec((BN, CW), row),
            pl.BlockSpec(w3.shape, full),
            pl.BlockSpec(b3.shape, full),
            pl.BlockSpec(w4.shape, full),
            pl.BlockSpec(b4.shape, full),
        ],
        out_specs=pl.BlockSpec((BN, DOUT), row),
        out_shape=jax.ShapeDtypeStruct((N_OBJ, DOUT), jnp.float32),
    )(pooled, cnt, w3, b3, w4, b4)


def kernel(obj_vecs, pred_vecs, edge_index, params):
    epad = jnp.full((E_PAD - N_PRED,), N_OBJ, jnp.int32)
    s_idx = jnp.concatenate([edge_index[0], epad])
    o_idx = jnp.concatenate([edge_index[1], epad])
    sidx_g = s_idx.reshape(NW, G_NCH, G_CH)
    oidx_g = o_idx.reshape(NW, G_NCH, G_CH)
    sidx_s = s_idx.reshape(NS, S_NCH, S_CH)
    oidx_s = o_idx.reshape(NS, S_NCH, S_CH)

    cnt = _counts(sidx_s, oidx_s)

    ov = obj_vecs
    pv = jnp.pad(pred_vecs, ((0, E_PAD - N_PRED), (0, 0)))
    for p in params:
        w1, b1, w2, b2, w3, b3, w4, b4 = p
        b1 = b1.reshape(1, -1)
        b2 = b2.reshape(1, -1)
        b3 = b3.reshape(1, -1)
        b4 = b4.reshape(1, -1)
        ovg = jnp.pad(ov, ((0, NODE_PAD - N_OBJ), (0, DOUT - ov.shape[1])))
        gs, go = _gather(ovg, sidx_g, oidx_g)
        ns, np_, no = _edge_mlp(gs, pv, go, w1, b1, w2, b2)
        pooled = _scatter(ns, no, sidx_s, oidx_s)
        ov = _node_mlp(pooled, cnt, w3, b3, w4, b4)
        pv = np_
    return ov, pv[:N_PRED]


# speedup vs baseline: 1.0061x; 1.0061x over previous
import functools

import jax
import jax.numpy as jnp
from jax import lax
from jax.experimental import pallas as pl
from jax.experimental.pallas import tpu as pltpu
from jax.experimental.pallas import tpu_sc as plsc

N_OBJ = 10000
N_PRED = 160000
H = 512
DOUT = 128

NC = 2
NS = 16
NW = NC * NS

E_PAD = 163840
NODE_PAD = 10016
A_ROWS = 10008

G_EPT = E_PAD // NW
G_CH = 128
G_NCH = G_EPT // G_CH

S_EPT = E_PAD // NS
S_CH = 40
S_NCH = S_EPT // S_CH
S_NPH = 2
S_PCH = S_NCH // S_NPH
S_K = 2
S_NV = S_PCH // S_K
SLAB = 128
RPT = 624
RPT_LAST = N_OBJ - (NS - 1) * RPT


def _sc_mesh():
    return plsc.VectorSubcoreMesh(core_axis_name="c", subcore_axis_name="s")


G_K = 2
G_HROWS = G_K * G_CH
G_NV = G_NCH // G_K


def _gather_body(d, table, sidx, oidx, gs_out, go_out, idx_v, stage, gsems,
                 osems):
    c = lax.axis_index("c")
    s = lax.axis_index("s")
    wid = s * NC + c
    base = wid * G_EPT

    def run(idx_hbm, out_hbm):
        pltpu.sync_copy(idx_hbm.at[wid], idx_v)

        def pair(dd, _):
            for h in range(2):
                v = 2 * dd + h

                @pl.when(v >= 2)
                def _():
                    pltpu.make_async_copy(
                        stage.at[h], out_hbm.at[pl.ds(base, G_HROWS)],
                        osems.at[h]).wait()

                for k in range(G_K):
                    pltpu.async_copy(
                        table.at[idx_v.at[v * G_K + k]],
                        stage.at[h, pl.ds(k * G_CH, G_CH)], gsems.at[h])
                for k in range(G_K):
                    pltpu.make_async_copy(
                        table.at[idx_v.at[0]],
                        stage.at[h, pl.ds(k * G_CH, G_CH)],
                        gsems.at[h]).wait()
                pltpu.async_copy(
                    stage.at[h],
                    out_hbm.at[pl.ds(base + v * G_HROWS, G_HROWS)],
                    osems.at[h])
            return ()

        lax.fori_loop(0, G_NV // 2, pair, (), unroll=False)
        for h in range(2):
            pltpu.make_async_copy(
                stage.at[h], out_hbm.at[pl.ds(base, G_HROWS)],
                osems.at[h]).wait()

    run(sidx, gs_out)
    run(oidx, go_out)


def _gather(table, sidx_g, oidx_g):
    d = table.shape[1]
    kfn = pl.kernel(
        functools.partial(_gather_body, d),
        out_type=(
            jax.ShapeDtypeStruct((E_PAD, d), jnp.float32),
            jax.ShapeDtypeStruct((E_PAD, d), jnp.float32),
        ),
        mesh=_sc_mesh(),
        scratch_types=[
            pltpu.VMEM((G_NCH, G_CH), jnp.int32),
            pltpu.VMEM((2, G_HROWS, d), jnp.float32),
            pltpu.SemaphoreType.DMA((2,)),
            pltpu.SemaphoreType.DMA((2,)),
        ],
    )
    return kfn(table, sidx_g, oidx_g)


def _fill(ref, rows, cols, value):
    v = jnp.full((16,), value, jnp.float32)

    def zrow(r, _):
        def zcol(k, _):
            ref[r, pl.ds(k * 16, 16)] = v
            return ()
        lax.fori_loop(0, cols // 16, zcol, (), unroll=True)
        return ()

    lax.fori_loop(0, rows, zrow, (), unroll=False)


def _own_rows(s, fn):
    @pl.when(s < NS - 1)
    def _():
        fn(s * RPT, RPT)

    @pl.when(s == NS - 1)
    def _():
        fn((NS - 1) * RPT, RPT_LAST)


def _zero_rows(acc, zbuf, s):
    def do(r0, n):
        def st(i, _):
            pltpu.sync_copy(zbuf, acc.at[pl.ds(r0 + i * 8, 8)])
            return ()
        lax.fori_loop(0, n // 8, st, (), unroll=False)
    _own_rows(s, do)


def _scatter_body(vs, vo, sidx, oidx, out, acc, idx_v, vals, zbuf,
                  rsems, ssems):
    c = lax.axis_index("c")
    s = lax.axis_index("s")
    _fill(zbuf, 8, SLAB, 0.0)

    for p in range(2):
        col0 = c * (2 * SLAB) + p * SLAB
        _zero_rows(acc, zbuf, s)
        plsc.subcore_barrier()
        for idx_hbm, val_hbm in ((sidx, vs), (oidx, vo)):
            for ph in range(S_NPH):
                pltpu.sync_copy(idx_hbm.at[s, pl.ds(ph * S_PCH, S_PCH)],
                                idx_v)
                ebase = s * S_EPT + ph * S_PCH * S_CH

                def pair(dd, _):
                    for h in range(2):
                        v = 2 * dd + h

                        @pl.when(v >= 2)
                        def _():
                            for k in range(S_K):
                                pltpu.make_async_copy(
                                    vals.at[h, pl.ds(k * S_CH, S_CH)],
                                    acc.at[pl.ds(0, S_CH)],
                                    ssems.at[h]).wait()

                        for k in range(S_K):
                            pltpu.async_copy(
                                val_hbm.at[
                                    pl.ds(ebase + (v * S_K + k) * S_CH,
                                          S_CH),
                                    pl.ds(col0, SLAB)],
                                vals.at[h, pl.ds(k * S_CH, S_CH)],
                                rsems.at[h])
                        for k in range(S_K):
                            pltpu.make_async_copy(
                                val_hbm.at[pl.ds(s * S_EPT, S_CH),
                                           pl.ds(col0, SLAB)],
                                vals.at[h, pl.ds(k * S_CH, S_CH)],
                                rsems.at[h]).wait()
                        for k in range(S_K):
                            pltpu.async_copy(
                                vals.at[h, pl.ds(k * S_CH, S_CH)],
                                acc.at[idx_v.at[v * S_K + k]], ssems.at[h],
                                add=True)
                    return ()

                lax.fori_loop(0, S_NV // 2, pair, (), unroll=False)
                for h in range(2):
                    for k in range(S_K):
                        pltpu.make_async_copy(
                            vals.at[h, pl.ds(k * S_CH, S_CH)],
                            acc.at[pl.ds(0, S_CH)], ssems.at[h]).wait()
        plsc.subcore_barrier()
        _own_rows(s, lambda r0, n: pltpu.sync_copy(
            acc.at[pl.ds(r0, n)],
            out.at[pl.ds(r0, n), pl.ds(col0, SLAB)]))


def _scatter(vs, vo, sidx_s, oidx_s):
    kfn = pl.kernel(
        _scatter_body,
        out_type=jax.ShapeDtypeStruct((N_OBJ, H), jnp.float32),
        mesh=_sc_mesh(),
        scratch_types=[
            pltpu.VMEM_SHARED((A_ROWS, SLAB), jnp.float32),
            pltpu.VMEM((S_PCH, S_CH), jnp.int32),
            pltpu.VMEM((2, S_K * S_CH, SLAB), jnp.float32),
            pltpu.VMEM((8, SLAB), jnp.float32),
            pltpu.SemaphoreType.DMA((2,)),
            pltpu.SemaphoreType.DMA((2,)),
        ],
    )
    return kfn(vs, vo, sidx_s, oidx_s)


CW = 128


def _counts_body(sidx, oidx, out, acc, idx_v, ones_v, zeros_v):
    c = lax.axis_index("c")
    s = lax.axis_index("s")

    @pl.when(c == 0)
    def _():
        _fill(ones_v, S_CH, CW, 1.0)
        _fill(zeros_v, 8, CW, 0.0)
        _zero_rows(acc, zeros_v, s)
        plsc.subcore_barrier()
        for idx_hbm in (sidx, oidx):
            pltpu.sync_copy(idx_hbm.at[s], idx_v)

            def step(j, _):
                pltpu.sync_copy(ones_v, acc.at[idx_v.at[j]], add=True)
                return ()

            lax.fori_loop(0, S_NCH, step, (), unroll=False)
        plsc.subcore_barrier()
        _own_rows(s, lambda r0, n: pltpu.sync_copy(
            acc.at[pl.ds(r0, n)], out.at[pl.ds(r0, n)]))


def _counts(sidx_s, oidx_s):
    kfn = pl.kernel(
        _counts_body,
        out_type=jax.ShapeDtypeStruct((N_OBJ, CW), jnp.float32),
        mesh=_sc_mesh(),
        scratch_types=[
            pltpu.VMEM_SHARED((A_ROWS, CW), jnp.float32),
            pltpu.VMEM((S_NCH, S_CH), jnp.int32),
            pltpu.VMEM((S_CH, CW), jnp.float32),
            pltpu.VMEM((8, CW), jnp.float32),
        ],
    )
    return kfn(sidx_s, oidx_s)


BE = 1024
BN = 1000


def _edge_mlp_body(din, gs, pred, go, w1, b1, w2, b2, ns, np_, no):
    h = jnp.dot(gs[:, :din], w1[:din, :], preferred_element_type=jnp.float32)
    h += jnp.dot(pred[...], w1[din:2 * din, :],
                 preferred_element_type=jnp.float32)
    h += jnp.dot(go[:, :din], w1[2 * din:, :],
                 preferred_element_type=jnp.float32)
    h = jax.nn.relu(h + b1[...])
    ns[...] = jax.nn.relu(
        jnp.dot(h, w2[:, :H], preferred_element_type=jnp.float32)
        + b2[:, :H])
    np_[...] = jax.nn.relu(
        jnp.dot(h, w2[:, H:H + DOUT], preferred_element_type=jnp.float32)
        + b2[:, H:H + DOUT])
    no[...] = jax.nn.relu(
        jnp.dot(h, w2[:, H + DOUT:], preferred_element_type=jnp.float32)
        + b2[:, H + DOUT:])


def _edge_mlp(gs, pred, go, w1, b1, w2, b2):
    din = w1.shape[0] // 3
    dg = gs.shape[1]
    grid = (E_PAD // BE,)
    row = lambda i: (i, 0)
    full = lambda i: (0, 0)
    return pl.pallas_call(
        functools.partial(_edge_mlp_body, din),
        grid=grid,
        in_specs=[
            pl.BlockSpec((BE, dg), row),
            pl.BlockSpec((BE, din), row),
            pl.BlockSpec((BE, dg), row),
            pl.BlockSpec(w1.shape, full),
            pl.BlockSpec(b1.shape, full),
            pl.BlockSpec(w2.shape, full),
            pl.BlockSpec(b2.shape, full),
        ],
        out_specs=[
            pl.BlockSpec((BE, H), row),
            pl.BlockSpec((BE, DOUT), row),
            pl.BlockSpec((BE, H), row),
        ],
        out_shape=[
            jax.ShapeDtypeStruct((E_PAD, H), jnp.float32),
            jax.ShapeDtypeStruct((E_PAD, DOUT), jnp.float32),
            jax.ShapeDtypeStruct((E_PAD, H), jnp.float32),
        ],
    )(gs, pred, go, w1, b1, w2, b2)


def _node_mlp_body(pooled, cnt, w3, b3, w4, b4, out):
    c = cnt[:, 0:1]
    inv = 1.0 / jnp.maximum(c, 1.0)
    h2 = jax.nn.relu(
        jnp.dot(pooled[...] * inv, w3[...], preferred_element_type=jnp.float32)
        + b3[...])
    out[...] = jnp.dot(h2, w4[...], preferred_element_type=jnp.float32) + b4[...]


def _node_mlp(pooled, cnt, w3, b3, w4, b4):
    grid = (N_OBJ // BN,)
    row = lambda i: (i, 0)
    full = lambda i: (0, 0)
    return pl.pallas_call(
        _node_mlp_body,
        grid=grid,
        in_specs=[
            pl.BlockSpec((BN, H), row),
            pl.BlockSpec((BN, CW), row),
            pl.BlockSpec(w3.shape, full),
            pl.BlockSpec(b3.shape, full),
            pl.BlockSpec(w4.shape, full),
            pl.BlockSpec(b4.shape, full),
        ],
        out_specs=pl.BlockSpec((BN, DOUT), row),
        out_shape=jax.ShapeDtypeStruct((N_OBJ, DOUT), jnp.float32),
    )(pooled, cnt, w3, b3, w4, b4)


def kernel(obj_vecs, pred_vecs, edge_index, params):
    epad = jnp.full((E_PAD - N_PRED,), N_OBJ, jnp.int32)
    s_idx = jnp.concatenate([edge_index[0], epad])
    o_idx = jnp.concatenate([edge_index[1], epad])
    sidx_g = s_idx.reshape(NW, G_NCH, G_CH)
    oidx_g = o_idx.reshape(NW, G_NCH, G_CH)
    sidx_s = s_idx.reshape(NS, S_NCH, S_CH)
    oidx_s = o_idx.reshape(NS, S_NCH, S_CH)

    cnt = _counts(sidx_s, oidx_s)

    ov = obj_vecs
    pv = jnp.pad(pred_vecs, ((0, E_PAD - N_PRED), (0, 0)))
    for p in params:
        w1, b1, w2, b2, w3, b3, w4, b4 = p
        b1 = b1.reshape(1, -1)
        b2 = b2.reshape(1, -1)
        b3 = b3.reshape(1, -1)
        b4 = b4.reshape(1, -1)
        ovg = jnp.pad(ov, ((0, NODE_PAD - N_OBJ), (0, DOUT - ov.shape[1])))
        gs, go = _gather(ovg, sidx_g, oidx_g)
        ns, np_, no = _edge_mlp(gs, pv, go, w1, b1, w2, b2)
        pooled = _scatter(ns, no, sidx_s, oidx_s)
        ov = _node_mlp(pooled, cnt, w3, b3, w4, b4)
        pv = np_
    return ov, pv[:N_PRED]


# speedup vs baseline: 1.0316x; 1.0254x over previous
import functools

import jax
import jax.numpy as jnp
from jax import lax
from jax.experimental import pallas as pl
from jax.experimental.pallas import tpu as pltpu
from jax.experimental.pallas import tpu_sc as plsc

N_OBJ = 10000
N_PRED = 160000
H = 512
DOUT = 128

NC = 2
NS = 16
NW = NC * NS

E_PAD = 163840
NODE_PAD = 10016
A_ROWS = 10008

G_EPT = E_PAD // NW
G_CH = 128
G_NCH = G_EPT // G_CH

S_EPT = E_PAD // NS
S_CH = 40
S_NCH = S_EPT // S_CH
S_NPH = 2
S_PCH = S_NCH // S_NPH
S_K = 2
S_NV = S_PCH // S_K
SLAB = 128
RPT = 624
RPT_LAST = N_OBJ - (NS - 1) * RPT


def _sc_mesh():
    return plsc.VectorSubcoreMesh(core_axis_name="c", subcore_axis_name="s")


G_NSL = 4


def _gather_body(d, table, sidx, oidx, gs_out, go_out, idx_v, stage, gsems,
                 osems):
    c = lax.axis_index("c")
    s = lax.axis_index("s")
    wid = s * NC + c
    base = wid * G_EPT

    def run(idx_hbm, out_hbm):
        pltpu.sync_copy(idx_hbm.at[wid], idx_v)

        def owait(q):
            pltpu.make_async_copy(
                stage.at[q], out_hbm.at[pl.ds(base, G_CH)],
                osems.at[q]).wait()

        def flush(u, q):
            pltpu.make_async_copy(
                table.at[idx_v.at[0]], stage.at[q], gsems.at[q]).wait()
            pltpu.async_copy(
                stage.at[q], out_hbm.at[pl.ds(base + u * G_CH, G_CH)],
                osems.at[q])

        def quad(dd, _):
            for q in range(G_NSL):
                v = dd * G_NSL + q

                @pl.when(dd >= 1)
                def _():
                    owait(q)

                pltpu.async_copy(table.at[idx_v.at[v]], stage.at[q],
                                 gsems.at[q])
                qq = (q + 1) % G_NSL
                if q == G_NSL - 1:
                    flush(v - (G_NSL - 1), qq)
                else:
                    @pl.when(dd >= 1)
                    def _():
                        flush(v - (G_NSL - 1), qq)
            return ()

        lax.fori_loop(0, G_NCH // G_NSL, quad, (), unroll=False)
        for u in range(G_NCH - (G_NSL - 1), G_NCH):
            flush(u, u % G_NSL)
        for q in range(G_NSL):
            owait(q)

    run(sidx, gs_out)
    run(oidx, go_out)


def _gather(table, sidx_g, oidx_g):
    d = table.shape[1]
    kfn = pl.kernel(
        functools.partial(_gather_body, d),
        out_type=(
            jax.ShapeDtypeStruct((E_PAD, d), jnp.float32),
            jax.ShapeDtypeStruct((E_PAD, d), jnp.float32),
        ),
        mesh=_sc_mesh(),
        scratch_types=[
            pltpu.VMEM((G_NCH, G_CH), jnp.int32),
            pltpu.VMEM((G_NSL, G_CH, d), jnp.float32),
            pltpu.SemaphoreType.DMA((G_NSL,)),
            pltpu.SemaphoreType.DMA((G_NSL,)),
        ],
    )
    return kfn(table, sidx_g, oidx_g)


def _fill(ref, rows, cols, value):
    v = jnp.full((16,), value, jnp.float32)

    def zrow(r, _):
        def zcol(k, _):
            ref[r, pl.ds(k * 16, 16)] = v
            return ()
        lax.fori_loop(0, cols // 16, zcol, (), unroll=True)
        return ()

    lax.fori_loop(0, rows, zrow, (), unroll=False)


def _own_rows(s, fn):
    @pl.when(s < NS - 1)
    def _():
        fn(s * RPT, RPT)

    @pl.when(s == NS - 1)
    def _():
        fn((NS - 1) * RPT, RPT_LAST)


def _zero_rows(acc, zbuf, s):
    def do(r0, n):
        def st(i, _):
            pltpu.sync_copy(zbuf, acc.at[pl.ds(r0 + i * 8, 8)])
            return ()
        lax.fori_loop(0, n // 8, st, (), unroll=False)
    _own_rows(s, do)


def _scatter_body(vs, vo, sidx, oidx, out, acc, idx_v, vals, zbuf,
                  rsems, ssems):
    c = lax.axis_index("c")
    s = lax.axis_index("s")
    _fill(zbuf, 8, SLAB, 0.0)

    for p in range(2):
        col0 = c * (2 * SLAB) + p * SLAB
        _zero_rows(acc, zbuf, s)
        plsc.subcore_barrier()
        for idx_hbm, val_hbm in ((sidx, vs), (oidx, vo)):
            for ph in range(S_NPH):
                pltpu.sync_copy(idx_hbm.at[s, pl.ds(ph * S_PCH, S_PCH)],
                                idx_v)
                ebase = s * S_EPT + ph * S_PCH * S_CH

                def pair(dd, _):
                    for h in range(2):
                        v = 2 * dd + h

                        @pl.when(v >= 2)
                        def _():
                            for k in range(S_K):
                                pltpu.make_async_copy(
                                    vals.at[h, pl.ds(k * S_CH, S_CH)],
                                    acc.at[pl.ds(0, S_CH)],
                                    ssems.at[h]).wait()

                        for k in range(S_K):
                            pltpu.async_copy(
                                val_hbm.at[
                                    pl.ds(ebase + (v * S_K + k) * S_CH,
                                          S_CH),
                                    pl.ds(col0, SLAB)],
                                vals.at[h, pl.ds(k * S_CH, S_CH)],
                                rsems.at[h])
                        for k in range(S_K):
                            pltpu.make_async_copy(
                                val_hbm.at[pl.ds(s * S_EPT, S_CH),
                                           pl.ds(col0, SLAB)],
                                vals.at[h, pl.ds(k * S_CH, S_CH)],
                                rsems.at[h]).wait()
                        for k in range(S_K):
                            pltpu.async_copy(
                                vals.at[h, pl.ds(k * S_CH, S_CH)],
                                acc.at[idx_v.at[v * S_K + k]], ssems.at[h],
                                add=True)
                    return ()

                lax.fori_loop(0, S_NV // 2, pair, (), unroll=False)
                for h in range(2):
                    for k in range(S_K):
                        pltpu.make_async_copy(
                            vals.at[h, pl.ds(k * S_CH, S_CH)],
                            acc.at[pl.ds(0, S_CH)], ssems.at[h]).wait()
        plsc.subcore_barrier()
        _own_rows(s, lambda r0, n: pltpu.sync_copy(
            acc.at[pl.ds(r0, n)],
            out.at[pl.ds(r0, n), pl.ds(col0, SLAB)]))


def _scatter(vs, vo, sidx_s, oidx_s):
    kfn = pl.kernel(
        _scatter_body,
        out_type=jax.ShapeDtypeStruct((N_OBJ, H), jnp.float32),
        mesh=_sc_mesh(),
        scratch_types=[
            pltpu.VMEM_SHARED((A_ROWS, SLAB), jnp.float32),
            pltpu.VMEM((S_PCH, S_CH), jnp.int32),
            pltpu.VMEM((2, S_K * S_CH, SLAB), jnp.float32),
            pltpu.VMEM((8, SLAB), jnp.float32),
            pltpu.SemaphoreType.DMA((2,)),
            pltpu.SemaphoreType.DMA((2,)),
        ],
    )
    return kfn(vs, vo, sidx_s, oidx_s)


CW = 128
C_CH = 64
C_NCH = S_EPT // C_CH


def _counts_body(sidx, oidx, out0, out1, acc, idx_v, ones_v, zeros_v, sem):
    c = lax.axis_index("c")
    s = lax.axis_index("s")
    _fill(ones_v, C_CH, CW, 1.0)
    _fill(zeros_v, 8, CW, 0.0)
    _zero_rows(acc, zeros_v, s)
    plsc.subcore_barrier()
    for cc, idx_hbm in ((0, sidx), (1, oidx)):
        @pl.when(c == cc)
        def _():
            pltpu.sync_copy(idx_hbm.at[s], idx_v)

            def fire(j, _):
                pltpu.async_copy(ones_v, acc.at[idx_v.at[j]], sem, add=True)
                return ()

            lax.fori_loop(0, C_NCH, fire, (), unroll=False)

            def drain(j, _):
                pltpu.make_async_copy(ones_v, acc.at[pl.ds(0, C_CH)],
                                      sem).wait()
                return ()

            lax.fori_loop(0, C_NCH, drain, (), unroll=False)
    plsc.subcore_barrier()
    for cc, out in ((0, out0), (1, out1)):
        @pl.when(c == cc)
        def _():
            _own_rows(s, lambda r0, n: pltpu.sync_copy(
                acc.at[pl.ds(r0, n)], out.at[pl.ds(r0, n)]))


def _counts(sidx_c, oidx_c):
    kfn = pl.kernel(
        _counts_body,
        out_type=(
            jax.ShapeDtypeStruct((N_OBJ, CW), jnp.float32),
            jax.ShapeDtypeStruct((N_OBJ, CW), jnp.float32),
        ),
        mesh=_sc_mesh(),
        scratch_types=[
            pltpu.VMEM_SHARED((A_ROWS, CW), jnp.float32),
            pltpu.VMEM((C_NCH, C_CH), jnp.int32),
            pltpu.VMEM((C_CH, CW), jnp.float32),
            pltpu.VMEM((8, CW), jnp.float32),
            pltpu.SemaphoreType.DMA,
        ],
    )
    return kfn(sidx_c, oidx_c)


BE = 1024
BN = 1000


def _edge_mlp_body(din, gs, pred, go, w1, b1, w2, b2, ns, np_, no):
    h = jnp.dot(gs[:, :din], w1[:din, :], preferred_element_type=jnp.float32)
    h += jnp.dot(pred[...], w1[din:2 * din, :],
                 preferred_element_type=jnp.float32)
    h += jnp.dot(go[:, :din], w1[2 * din:, :],
                 preferred_element_type=jnp.float32)
    h = jax.nn.relu(h + b1[...])
    ns[...] = jax.nn.relu(
        jnp.dot(h, w2[:, :H], preferred_element_type=jnp.float32)
        + b2[:, :H])
    np_[...] = jax.nn.relu(
        jnp.dot(h, w2[:, H:H + DOUT], preferred_element_type=jnp.float32)
        + b2[:, H:H + DOUT])
    no[...] = jax.nn.relu(
        jnp.dot(h, w2[:, H + DOUT:], preferred_element_type=jnp.float32)
        + b2[:, H + DOUT:])


def _edge_mlp(gs, pred, go, w1, b1, w2, b2):
    din = w1.shape[0] // 3
    dg = gs.shape[1]
    grid = (E_PAD // BE,)
    row = lambda i: (i, 0)
    full = lambda i: (0, 0)
    return pl.pallas_call(
        functools.partial(_edge_mlp_body, din),
        grid=grid,
        in_specs=[
            pl.BlockSpec((BE, dg), row),
            pl.BlockSpec((BE, din), row),
            pl.BlockSpec((BE, dg), row),
            pl.BlockSpec(w1.shape, full),
            pl.BlockSpec(b1.shape, full),
            pl.BlockSpec(w2.shape, full),
            pl.BlockSpec(b2.shape, full),
        ],
        out_specs=[
            pl.BlockSpec((BE, H), row),
            pl.BlockSpec((BE, DOUT), row),
            pl.BlockSpec((BE, H), row),
        ],
        out_shape=[
            jax.ShapeDtypeStruct((E_PAD, H), jnp.float32),
            jax.ShapeDtypeStruct((E_PAD, DOUT), jnp.float32),
            jax.ShapeDtypeStruct((E_PAD, H), jnp.float32),
        ],
    )(gs, pred, go, w1, b1, w2, b2)


def _node_mlp_body(pooled, cnt0, cnt1, w3, b3, w4, b4, out):
    c = cnt0[:, 0:1] + cnt1[:, 0:1]
    inv = 1.0 / jnp.maximum(c, 1.0)
    h2 = jax.nn.relu(
        jnp.dot(pooled[...] * inv, w3[...], preferred_element_type=jnp.float32)
        + b3[...])
    out[...] = jnp.dot(h2, w4[...], preferred_element_type=jnp.float32) + b4[...]


def _node_mlp(pooled, cnt0, cnt1, w3, b3, w4, b4):
    grid = (N_OBJ // BN,)
    row = lambda i: (i, 0)
    full = lambda i: (0, 0)
    return pl.pallas_call(
        _node_mlp_body,
        grid=grid,
        in_specs=[
            pl.BlockSpec((BN, H), row),
            pl.BlockSpec((BN, CW), row),
            pl.BlockSpec((BN, CW), row),
            pl.BlockSpec(w3.shape, full),
            pl.BlockSpec(b3.shape, full),
            pl.BlockSpec(w4.shape, full),
            pl.BlockSpec(b4.shape, full),
        ],
        out_specs=pl.BlockSpec((BN, DOUT), row),
        out_shape=jax.ShapeDtypeStruct((N_OBJ, DOUT), jnp.float32),
    )(pooled, cnt0, cnt1, w3, b3, w4, b4)


def kernel(obj_vecs, pred_vecs, edge_index, params):
    epad = jnp.full((E_PAD - N_PRED,), N_OBJ, jnp.int32)
    s_idx = jnp.concatenate([edge_index[0], epad])
    o_idx = jnp.concatenate([edge_index[1], epad])
    sidx_g = s_idx.reshape(NW, G_NCH, G_CH)
    oidx_g = o_idx.reshape(NW, G_NCH, G_CH)
    sidx_s = s_idx.reshape(NS, S_NCH, S_CH)
    oidx_s = o_idx.reshape(NS, S_NCH, S_CH)
    sidx_c = s_idx.reshape(NS, C_NCH, C_CH)
    oidx_c = o_idx.reshape(NS, C_NCH, C_CH)

    cnt0, cnt1 = _counts(sidx_c, oidx_c)

    ov = obj_vecs
    pv = jnp.pad(pred_vecs, ((0, E_PAD - N_PRED), (0, 0)))
    for p in params:
        w1, b1, w2, b2, w3, b3, w4, b4 = p
        b1 = b1.reshape(1, -1)
        b2 = b2.reshape(1, -1)
        b3 = b3.reshape(1, -1)
        b4 = b4.reshape(1, -1)
        ovg = jnp.pad(ov, ((0, NODE_PAD - N_OBJ), (0, DOUT - ov.shape[1])))
        gs, go = _gather(ovg, sidx_g, oidx_g)
        ns, np_, no = _edge_mlp(gs, pv, go, w1, b1, w2, b2)
        pooled = _scatter(ns, no, sidx_s, oidx_s)
        ov = _node_mlp(pooled, cnt0, cnt1, w3, b3, w4, b4)
        pv = np_
    return ov, pv[:N_PRED]


# speedup vs baseline: 1.0403x; 1.0084x over previous
import functools

import jax
import jax.numpy as jnp
from jax import lax
from jax.experimental import pallas as pl
from jax.experimental.pallas import tpu as pltpu
from jax.experimental.pallas import tpu_sc as plsc

N_OBJ = 10000
N_PRED = 160000
H = 512
DOUT = 128

NC = 2
NS = 16
NW = NC * NS

E_PAD = 163840
A_ROWS = 10008

G_EPT = E_PAD // NW
G_CH = 128
G_NCH = G_EPT // G_CH

S_EPT = E_PAD // NS
S_CH = 40
S_NCH = S_EPT // S_CH
S_NPH = 2
S_PCH = S_NCH // S_NPH
S_K = 2
S_NV = S_PCH // S_K
SLAB = 128
RPT = 624
RPT_LAST = N_OBJ - (NS - 1) * RPT


def _sc_mesh():
    return plsc.VectorSubcoreMesh(core_axis_name="c", subcore_axis_name="s")


G_NSL = 4


def _gather_body(d, table, sidx, oidx, gs_out, go_out, idx_v, stage, gsems,
                 osems):
    c = lax.axis_index("c")
    s = lax.axis_index("s")
    wid = s * NC + c
    base = wid * G_EPT

    def run(idx_hbm, out_hbm):
        pltpu.sync_copy(idx_hbm.at[wid], idx_v)

        def owait(q):
            pltpu.make_async_copy(
                stage.at[q], out_hbm.at[pl.ds(base, G_CH)],
                osems.at[q]).wait()

        def flush(u, q):
            pltpu.make_async_copy(
                table.at[idx_v.at[0]], stage.at[q], gsems.at[q]).wait()
            pltpu.async_copy(
                stage.at[q], out_hbm.at[pl.ds(base + u * G_CH, G_CH)],
                osems.at[q])

        def quad(dd, _):
            for q in range(G_NSL):
                v = dd * G_NSL + q

                @pl.when(dd >= 1)
                def _():
                    owait(q)

                pltpu.async_copy(table.at[idx_v.at[v]], stage.at[q],
                                 gsems.at[q])
                qq = (q + 1) % G_NSL
                if q == G_NSL - 1:
                    flush(v - (G_NSL - 1), qq)
                else:
                    @pl.when(dd >= 1)
                    def _():
                        flush(v - (G_NSL - 1), qq)
            return ()

        lax.fori_loop(0, G_NCH // G_NSL, quad, (), unroll=False)
        for u in range(G_NCH - (G_NSL - 1), G_NCH):
            flush(u, u % G_NSL)
        for q in range(G_NSL):
            owait(q)

    run(sidx, gs_out)
    run(oidx, go_out)


def _gather(table, sidx_g, oidx_g):
    d = table.shape[1]
    kfn = pl.kernel(
        functools.partial(_gather_body, d),
        out_type=(
            jax.ShapeDtypeStruct((E_PAD, d), jnp.float32),
            jax.ShapeDtypeStruct((E_PAD, d), jnp.float32),
        ),
        mesh=_sc_mesh(),
        scratch_types=[
            pltpu.VMEM((G_NCH, G_CH), jnp.int32),
            pltpu.VMEM((G_NSL, G_CH, d), jnp.float32),
            pltpu.SemaphoreType.DMA((G_NSL,)),
            pltpu.SemaphoreType.DMA((G_NSL,)),
        ],
    )
    return kfn(table, sidx_g, oidx_g)


def _fill(ref, rows, cols, value):
    v = jnp.full((16,), value, jnp.float32)

    def zrow(r, _):
        def zcol(k, _):
            ref[r, pl.ds(k * 16, 16)] = v
            return ()
        lax.fori_loop(0, cols // 16, zcol, (), unroll=True)
        return ()

    lax.fori_loop(0, rows, zrow, (), unroll=False)


def _own_rows(s, fn):
    @pl.when(s < NS - 1)
    def _():
        fn(s * RPT, RPT)

    @pl.when(s == NS - 1)
    def _():
        fn((NS - 1) * RPT, RPT_LAST)


def _zero_rows(acc, zbuf, s):
    def do(r0, n):
        def st(i, _):
            pltpu.sync_copy(zbuf, acc.at[pl.ds(r0 + i * 8, 8)])
            return ()
        lax.fori_loop(0, n // 8, st, (), unroll=False)
    _own_rows(s, do)


def _scatter_body(vs, vo, sidx, oidx, out, acc, idx_v, vals, zbuf,
                  rsems, ssems):
    c = lax.axis_index("c")
    s = lax.axis_index("s")
    _fill(zbuf, 8, SLAB, 0.0)

    for p in range(2):
        col0 = c * (2 * SLAB) + p * SLAB
        _zero_rows(acc, zbuf, s)
        plsc.subcore_barrier()
        for idx_hbm, val_hbm in ((sidx, vs), (oidx, vo)):
            for ph in range(S_NPH):
                pltpu.sync_copy(idx_hbm.at[s, pl.ds(ph * S_PCH, S_PCH)],
                                idx_v)
                ebase = s * S_EPT + ph * S_PCH * S_CH

                def pair(dd, _):
                    for h in range(2):
                        v = 2 * dd + h

                        @pl.when(v >= 2)
                        def _():
                            for k in range(S_K):
                                pltpu.make_async_copy(
                                    vals.at[h, pl.ds(k * S_CH, S_CH)],
                                    acc.at[pl.ds(0, S_CH)],
                                    ssems.at[h]).wait()

                        for k in range(S_K):
                            pltpu.async_copy(
                                val_hbm.at[
                                    pl.ds(ebase + (v * S_K + k) * S_CH,
                                          S_CH),
                                    pl.ds(col0, SLAB)],
                                vals.at[h, pl.ds(k * S_CH, S_CH)],
                                rsems.at[h])
                        for k in range(S_K):
                            pltpu.make_async_copy(
                                val_hbm.at[pl.ds(s * S_EPT, S_CH),
                                           pl.ds(col0, SLAB)],
                                vals.at[h, pl.ds(k * S_CH, S_CH)],
                                rsems.at[h]).wait()
                        for k in range(S_K):
                            pltpu.async_copy(
                                vals.at[h, pl.ds(k * S_CH, S_CH)],
                                acc.at[idx_v.at[v * S_K + k]], ssems.at[h],
                                add=True)
                    return ()

                lax.fori_loop(0, S_NV // 2, pair, (), unroll=False)
                for h in range(2):
                    for k in range(S_K):
                        pltpu.make_async_copy(
                            vals.at[h, pl.ds(k * S_CH, S_CH)],
                            acc.at[pl.ds(0, S_CH)], ssems.at[h]).wait()
        plsc.subcore_barrier()
        _own_rows(s, lambda r0, n: pltpu.sync_copy(
            acc.at[pl.ds(r0, n)],
            out.at[pl.ds(r0, n), pl.ds(col0, SLAB)]))


def _scatter(vs, vo, sidx_s, oidx_s):
    kfn = pl.kernel(
        _scatter_body,
        out_type=jax.ShapeDtypeStruct((N_OBJ, H), jnp.float32),
        mesh=_sc_mesh(),
        scratch_types=[
            pltpu.VMEM_SHARED((A_ROWS, SLAB), jnp.float32),
            pltpu.VMEM((S_PCH, S_CH), jnp.int32),
            pltpu.VMEM((2, S_K * S_CH, SLAB), jnp.float32),
            pltpu.VMEM((8, SLAB), jnp.float32),
            pltpu.SemaphoreType.DMA((2,)),
            pltpu.SemaphoreType.DMA((2,)),
        ],
    )
    return kfn(vs, vo, sidx_s, oidx_s)


CW = 128
C_CH = 64
C_NCH = S_EPT // C_CH


def _counts_body(sidx, oidx, out0, out1, acc, idx_v, ones_v, zeros_v, sem):
    c = lax.axis_index("c")
    s = lax.axis_index("s")
    _fill(ones_v, C_CH, CW, 1.0)
    _fill(zeros_v, 8, CW, 0.0)
    _zero_rows(acc, zeros_v, s)
    plsc.subcore_barrier()
    for cc, idx_hbm in ((0, sidx), (1, oidx)):
        @pl.when(c == cc)
        def _():
            pltpu.sync_copy(idx_hbm.at[s], idx_v)

            def fire(j, _):
                pltpu.async_copy(ones_v, acc.at[idx_v.at[j]], sem, add=True)
                return ()

            lax.fori_loop(0, C_NCH, fire, (), unroll=False)

            def drain(j, _):
                pltpu.make_async_copy(ones_v, acc.at[pl.ds(0, C_CH)],
                                      sem).wait()
                return ()

            lax.fori_loop(0, C_NCH, drain, (), unroll=False)
    plsc.subcore_barrier()
    for cc, out in ((0, out0), (1, out1)):
        @pl.when(c == cc)
        def _():
            _own_rows(s, lambda r0, n: pltpu.sync_copy(
                acc.at[pl.ds(r0, n)], out.at[pl.ds(r0, n)]))


def _counts(sidx_c, oidx_c):
    kfn = pl.kernel(
        _counts_body,
        out_type=(
            jax.ShapeDtypeStruct((N_OBJ, CW), jnp.float32),
            jax.ShapeDtypeStruct((N_OBJ, CW), jnp.float32),
        ),
        mesh=_sc_mesh(),
        scratch_types=[
            pltpu.VMEM_SHARED((A_ROWS, CW), jnp.float32),
            pltpu.VMEM((C_NCH, C_CH), jnp.int32),
            pltpu.VMEM((C_CH, CW), jnp.float32),
            pltpu.VMEM((8, CW), jnp.float32),
            pltpu.SemaphoreType.DMA,
        ],
    )
    return kfn(sidx_c, oidx_c)


BE = 1024
BN = 1000


def _edge_mlp_body(din, gs, pred, go, w1, b1, w2, b2, ns, np_, no):
    h = jnp.dot(gs[:, :din], w1[:din, :], preferred_element_type=jnp.float32)
    h += jnp.dot(pred[...], w1[din:2 * din, :],
                 preferred_element_type=jnp.float32)
    h += jnp.dot(go[:, :din], w1[2 * din:, :],
                 preferred_element_type=jnp.float32)
    h = jax.nn.relu(h + b1[...])
    ns[...] = jax.nn.relu(
        jnp.dot(h, w2[:, :H], preferred_element_type=jnp.float32)
        + b2[:, :H])
    np_[...] = jax.nn.relu(
        jnp.dot(h, w2[:, H:H + DOUT], preferred_element_type=jnp.float32)
        + b2[:, H:H + DOUT])
    no[...] = jax.nn.relu(
        jnp.dot(h, w2[:, H + DOUT:], preferred_element_type=jnp.float32)
        + b2[:, H + DOUT:])


def _edge_mlp(gs, pred, go, w1, b1, w2, b2):
    din = w1.shape[0] // 3
    dg = gs.shape[1]
    grid = (E_PAD // BE,)
    row = lambda i: (i, 0)
    full = lambda i: (0, 0)
    return pl.pallas_call(
        functools.partial(_edge_mlp_body, din),
        grid=grid,
        in_specs=[
            pl.BlockSpec((BE, dg), row),
            pl.BlockSpec((BE, din), row),
            pl.BlockSpec((BE, dg), row),
            pl.BlockSpec(w1.shape, full),
            pl.BlockSpec(b1.shape, full),
            pl.BlockSpec(w2.shape, full),
            pl.BlockSpec(b2.shape, full),
        ],
        out_specs=[
            pl.BlockSpec((BE, H), row),
            pl.BlockSpec((BE, DOUT), row),
            pl.BlockSpec((BE, H), row),
        ],
        out_shape=[
            jax.ShapeDtypeStruct((E_PAD, H), jnp.float32),
            jax.ShapeDtypeStruct((E_PAD, DOUT), jnp.float32),
            jax.ShapeDtypeStruct((E_PAD, H), jnp.float32),
        ],
    )(gs, pred, go, w1, b1, w2, b2)


def _node_mlp_body(pooled, cnt0, cnt1, w3, b3, w4, b4, out):
    c = cnt0[:, 0:1] + cnt1[:, 0:1]
    inv = 1.0 / jnp.maximum(c, 1.0)
    h2 = jax.nn.relu(
        jnp.dot(pooled[...] * inv, w3[...], preferred_element_type=jnp.float32)
        + b3[...])
    out[...] = jnp.dot(h2, w4[...], preferred_element_type=jnp.float32) + b4[...]


def _node_mlp(pooled, cnt0, cnt1, w3, b3, w4, b4):
    grid = (N_OBJ // BN,)
    row = lambda i: (i, 0)
    full = lambda i: (0, 0)
    return pl.pallas_call(
        _node_mlp_body,
        grid=grid,
        in_specs=[
            pl.BlockSpec((BN, H), row),
            pl.BlockSpec((BN, CW), row),
            pl.BlockSpec((BN, CW), row),
            pl.BlockSpec(w3.shape, full),
            pl.BlockSpec(b3.shape, full),
            pl.BlockSpec(w4.shape, full),
            pl.BlockSpec(b4.shape, full),
        ],
        out_specs=pl.BlockSpec((BN, DOUT), row),
        out_shape=jax.ShapeDtypeStruct((N_OBJ, DOUT), jnp.float32),
    )(pooled, cnt0, cnt1, w3, b3, w4, b4)


def kernel(obj_vecs, pred_vecs, edge_index, params):
    epad = jnp.full((E_PAD - N_PRED,), N_OBJ, jnp.int32)
    zpad = jnp.zeros((E_PAD - N_PRED,), jnp.int32)
    s_idx = jnp.concatenate([edge_index[0], epad])
    o_idx = jnp.concatenate([edge_index[1], epad])
    sg = jnp.concatenate([edge_index[0], zpad])
    og = jnp.concatenate([edge_index[1], zpad])
    sidx_g = sg.reshape(NW, G_NCH, G_CH)
    oidx_g = og.reshape(NW, G_NCH, G_CH)
    sidx_s = s_idx.reshape(NS, S_NCH, S_CH)
    oidx_s = o_idx.reshape(NS, S_NCH, S_CH)
    sidx_c = s_idx.reshape(NS, C_NCH, C_CH)
    oidx_c = o_idx.reshape(NS, C_NCH, C_CH)

    cnt0, cnt1 = _counts(sidx_c, oidx_c)

    ov = obj_vecs
    pv = jnp.pad(pred_vecs, ((0, E_PAD - N_PRED), (0, 0)))
    for p in params:
        w1, b1, w2, b2, w3, b3, w4, b4 = p
        b1 = b1.reshape(1, -1)
        b2 = b2.reshape(1, -1)
        b3 = b3.reshape(1, -1)
        b4 = b4.reshape(1, -1)
        ovg = ov
        if ovg.shape[1] < DOUT:
            ovg = jnp.pad(ovg, ((0, 0), (0, DOUT - ovg.shape[1])))
        gs, go = _gather(ovg, sidx_g, oidx_g)
        ns, np_, no = _edge_mlp(gs, pv, go, w1, b1, w2, b2)
        pooled = _scatter(ns, no, sidx_s, oidx_s)
        ov = _node_mlp(pooled, cnt0, cnt1, w3, b3, w4, b4)
        pv = np_
    return ov, pv[:N_PRED]


# speedup vs baseline: 1.1924x; 1.1462x over previous
import functools

import jax
import jax.numpy as jnp
from jax import lax
from jax.experimental import pallas as pl
from jax.experimental.pallas import tpu as pltpu
from jax.experimental.pallas import tpu_sc as plsc

N_OBJ = 10000
N_PRED = 160000
H = 512
DOUT = 128

NC = 2
NS = 16
NW = NC * NS

E_PAD = 163840
A_ROWS = 10008

G_EPT = E_PAD // NW
G_CH = 128
G_NCH = G_EPT // G_CH

S_EPT = E_PAD // NS
S_CH = 40
S_NCH = S_EPT // S_CH
S_NPH = 4
S_PCH = S_NCH // S_NPH
S_NSL = 4
SLAB = 128
RPT = 624
RPT_LAST = N_OBJ - (NS - 1) * RPT


def _sc_mesh():
    return plsc.VectorSubcoreMesh(core_axis_name="c", subcore_axis_name="s")


G_NSL = 4


def _gather_body(d, table, sidx, oidx, gs_out, go_out, idx_v, stage, gsems,
                 osems):
    c = lax.axis_index("c")
    s = lax.axis_index("s")
    wid = s * NC + c
    base = wid * G_EPT

    def run(idx_hbm, out_hbm):
        pltpu.sync_copy(idx_hbm.at[wid], idx_v)

        def owait(q):
            pltpu.make_async_copy(
                stage.at[q], out_hbm.at[pl.ds(base, G_CH)],
                osems.at[q]).wait()

        def flush(u, q):
            pltpu.make_async_copy(
                table.at[idx_v.at[0]], stage.at[q], gsems.at[q]).wait()
            pltpu.async_copy(
                stage.at[q], out_hbm.at[pl.ds(base + u * G_CH, G_CH)],
                osems.at[q])

        def quad(dd, _):
            for q in range(G_NSL):
                v = dd * G_NSL + q

                @pl.when(dd >= 1)
                def _():
                    owait(q)

                pltpu.async_copy(table.at[idx_v.at[v]], stage.at[q],
                                 gsems.at[q])
                qq = (q + 1) % G_NSL
                if q == G_NSL - 1:
                    flush(v - (G_NSL - 1), qq)
                else:
                    @pl.when(dd >= 1)
                    def _():
                        flush(v - (G_NSL - 1), qq)
            return ()

        lax.fori_loop(0, G_NCH // G_NSL, quad, (), unroll=False)
        for u in range(G_NCH - (G_NSL - 1), G_NCH):
            flush(u, u % G_NSL)
        for q in range(G_NSL):
            owait(q)

    run(sidx, gs_out)
    run(oidx, go_out)


def _gather(table, sidx_g, oidx_g):
    d = table.shape[1]
    kfn = pl.kernel(
        functools.partial(_gather_body, d),
        out_type=(
            jax.ShapeDtypeStruct((E_PAD, d), jnp.float32),
            jax.ShapeDtypeStruct((E_PAD, d), jnp.float32),
        ),
        mesh=_sc_mesh(),
        scratch_types=[
            pltpu.VMEM((G_NCH, G_CH), jnp.int32),
            pltpu.VMEM((G_NSL, G_CH, d), jnp.float32),
            pltpu.SemaphoreType.DMA((G_NSL,)),
            pltpu.SemaphoreType.DMA((G_NSL,)),
        ],
    )
    return kfn(table, sidx_g, oidx_g)


def _fill(ref, rows, cols, value):
    v = jnp.full((16,), value, jnp.float32)

    def zrow(r, _):
        def zcol(k, _):
            ref[r, pl.ds(k * 16, 16)] = v
            return ()
        lax.fori_loop(0, cols // 16, zcol, (), unroll=True)
        return ()

    lax.fori_loop(0, rows, zrow, (), unroll=False)


def _own_rows(s, fn):
    @pl.when(s < NS - 1)
    def _():
        fn(s * RPT, RPT)

    @pl.when(s == NS - 1)
    def _():
        fn((NS - 1) * RPT, RPT_LAST)


def _zero_rows(acc, zbuf, s):
    def do(r0, n):
        def st(i, _):
            pltpu.sync_copy(zbuf, acc.at[pl.ds(r0 + i * 8, 8)])
            return ()
        lax.fori_loop(0, n // 8, st, (), unroll=False)
    _own_rows(s, do)


def _scatter_body(vs, vo, sidx, oidx, out, acc, idx_v, vals, zbuf,
                  rsems, ssems):
    c = lax.axis_index("c")
    s = lax.axis_index("s")
    _fill(zbuf, 8, SLAB, 0.0)

    for p in range(2):
        col0 = c * (2 * SLAB) + p * SLAB
        _zero_rows(acc, zbuf, s)
        plsc.subcore_barrier()
        for idx_hbm, val_hbm in ((sidx, vs), (oidx, vo)):
            for ph in range(S_NPH):
                pltpu.sync_copy(idx_hbm.at[s, pl.ds(ph * S_PCH, S_PCH)],
                                idx_v)
                ebase = s * S_EPT + ph * S_PCH * S_CH

                def swait(q):
                    pltpu.make_async_copy(
                        vals.at[q], acc.at[pl.ds(0, S_CH)],
                        ssems.at[q]).wait()

                def add(u, q):
                    pltpu.make_async_copy(
                        val_hbm.at[pl.ds(ebase, S_CH), pl.ds(col0, SLAB)],
                        vals.at[q], rsems.at[q]).wait()
                    pltpu.async_copy(vals.at[q], acc.at[idx_v.at[u]],
                                     ssems.at[q], add=True)

                def quad(dd, _):
                    for q in range(S_NSL):
                        v = dd * S_NSL + q

                        @pl.when(dd >= 1)
                        def _():
                            swait(q)

                        pltpu.async_copy(
                            val_hbm.at[pl.ds(ebase + v * S_CH, S_CH),
                                       pl.ds(col0, SLAB)],
                            vals.at[q], rsems.at[q])
                        qq = (q + 1) % S_NSL
                        if q == S_NSL - 1:
                            add(v - (S_NSL - 1), qq)
                        else:
                            @pl.when(dd >= 1)
                            def _():
                                add(v - (S_NSL - 1), qq)
                    return ()

                lax.fori_loop(0, S_PCH // S_NSL, quad, (), unroll=False)
                for u in range(S_PCH - (S_NSL - 1), S_PCH):
                    add(u, u % S_NSL)
                for q in range(S_NSL):
                    swait(q)
        plsc.subcore_barrier()
        _own_rows(s, lambda r0, n: pltpu.sync_copy(
            acc.at[pl.ds(r0, n)],
            out.at[pl.ds(r0, n), pl.ds(col0, SLAB)]))


def _scatter(vs, vo, sidx_s, oidx_s):
    kfn = pl.kernel(
        _scatter_body,
        out_type=jax.ShapeDtypeStruct((N_OBJ, H), jnp.float32),
        mesh=_sc_mesh(),
        scratch_types=[
            pltpu.VMEM_SHARED((A_ROWS, SLAB), jnp.float32),
            pltpu.VMEM((S_PCH, S_CH), jnp.int32),
            pltpu.VMEM((S_NSL, S_CH, SLAB), jnp.float32),
            pltpu.VMEM((8, SLAB), jnp.float32),
            pltpu.SemaphoreType.DMA((S_NSL,)),
            pltpu.SemaphoreType.DMA((S_NSL,)),
        ],
    )
    return kfn(vs, vo, sidx_s, oidx_s)


CW = 128
C_CH = 64
C_NCH = S_EPT // C_CH


def _counts_body(sidx, oidx, out0, out1, acc, idx_v, ones_v, zeros_v, sem):
    c = lax.axis_index("c")
    s = lax.axis_index("s")
    _fill(ones_v, C_CH, CW, 1.0)
    _fill(zeros_v, 8, CW, 0.0)
    _zero_rows(acc, zeros_v, s)
    plsc.subcore_barrier()
    for cc, idx_hbm in ((0, sidx), (1, oidx)):
        @pl.when(c == cc)
        def _():
            pltpu.sync_copy(idx_hbm.at[s], idx_v)

            def fire(j, _):
                pltpu.async_copy(ones_v, acc.at[idx_v.at[j]], sem, add=True)
                return ()

            lax.fori_loop(0, C_NCH, fire, (), unroll=False)

            def drain(j, _):
                pltpu.make_async_copy(ones_v, acc.at[pl.ds(0, C_CH)],
                                      sem).wait()
                return ()

            lax.fori_loop(0, C_NCH, drain, (), unroll=False)
    plsc.subcore_barrier()
    for cc, out in ((0, out0), (1, out1)):
        @pl.when(c == cc)
        def _():
            _own_rows(s, lambda r0, n: pltpu.sync_copy(
                acc.at[pl.ds(r0, n)], out.at[pl.ds(r0, n)]))


def _counts(sidx_c, oidx_c):
    kfn = pl.kernel(
        _counts_body,
        out_type=(
            jax.ShapeDtypeStruct((N_OBJ, CW), jnp.float32),
            jax.ShapeDtypeStruct((N_OBJ, CW), jnp.float32),
        ),
        mesh=_sc_mesh(),
        scratch_types=[
            pltpu.VMEM_SHARED((A_ROWS, CW), jnp.float32),
            pltpu.VMEM((C_NCH, C_CH), jnp.int32),
            pltpu.VMEM((C_CH, CW), jnp.float32),
            pltpu.VMEM((8, CW), jnp.float32),
            pltpu.SemaphoreType.DMA,
        ],
    )
    return kfn(sidx_c, oidx_c)


BE = 1024
BN = 1000


def _edge_mlp_body(din, gs, pred, go, w1, b1, w2, b2, ns, np_, no):
    h = jnp.dot(gs[:, :din], w1[:din, :], preferred_element_type=jnp.float32)
    h += jnp.dot(pred[...], w1[din:2 * din, :],
                 preferred_element_type=jnp.float32)
    h += jnp.dot(go[:, :din], w1[2 * din:, :],
                 preferred_element_type=jnp.float32)
    h = jax.nn.relu(h + b1[...])
    ns[...] = jax.nn.relu(
        jnp.dot(h, w2[:, :H], preferred_element_type=jnp.float32)
        + b2[:, :H])
    np_[...] = jax.nn.relu(
        jnp.dot(h, w2[:, H:H + DOUT], preferred_element_type=jnp.float32)
        + b2[:, H:H + DOUT])
    no[...] = jax.nn.relu(
        jnp.dot(h, w2[:, H + DOUT:], preferred_element_type=jnp.float32)
        + b2[:, H + DOUT:])


def _edge_mlp(gs, pred, go, w1, b1, w2, b2):
    din = w1.shape[0] // 3
    dg = gs.shape[1]
    grid = (E_PAD // BE,)
    row = lambda i: (i, 0)
    full = lambda i: (0, 0)
    return pl.pallas_call(
        functools.partial(_edge_mlp_body, din),
        grid=grid,
        in_specs=[
            pl.BlockSpec((BE, dg), row),
            pl.BlockSpec((BE, din), row),
            pl.BlockSpec((BE, dg), row),
            pl.BlockSpec(w1.shape, full),
            pl.BlockSpec(b1.shape, full),
            pl.BlockSpec(w2.shape, full),
            pl.BlockSpec(b2.shape, full),
        ],
        out_specs=[
            pl.BlockSpec((BE, H), row),
            pl.BlockSpec((BE, DOUT), row),
            pl.BlockSpec((BE, H), row),
        ],
        out_shape=[
            jax.ShapeDtypeStruct((E_PAD, H), jnp.float32),
            jax.ShapeDtypeStruct((E_PAD, DOUT), jnp.float32),
            jax.ShapeDtypeStruct((E_PAD, H), jnp.float32),
        ],
    )(gs, pred, go, w1, b1, w2, b2)


def _node_mlp_body(pooled, cnt0, cnt1, w3, b3, w4, b4, out):
    c = cnt0[:, 0:1] + cnt1[:, 0:1]
    inv = 1.0 / jnp.maximum(c, 1.0)
    h2 = jax.nn.relu(
        jnp.dot(pooled[...] * inv, w3[...], preferred_element_type=jnp.float32)
        + b3[...])
    out[...] = jnp.dot(h2, w4[...], preferred_element_type=jnp.float32) + b4[...]


def _node_mlp(pooled, cnt0, cnt1, w3, b3, w4, b4):
    grid = (N_OBJ // BN,)
    row = lambda i: (i, 0)
    full = lambda i: (0, 0)
    return pl.pallas_call(
        _node_mlp_body,
        grid=grid,
        in_specs=[
            pl.BlockSpec((BN, H), row),
            pl.BlockSpec((BN, CW), row),
            pl.BlockSpec((BN, CW), row),
            pl.BlockSpec(w3.shape, full),
            pl.BlockSpec(b3.shape, full),
            pl.BlockSpec(w4.shape, full),
            pl.BlockSpec(b4.shape, full),
        ],
        out_specs=pl.BlockSpec((BN, DOUT), row),
        out_shape=jax.ShapeDtypeStruct((N_OBJ, DOUT), jnp.float32),
    )(pooled, cnt0, cnt1, w3, b3, w4, b4)


def kernel(obj_vecs, pred_vecs, edge_index, params):
    epad = jnp.full((E_PAD - N_PRED,), N_OBJ, jnp.int32)
    zpad = jnp.zeros((E_PAD - N_PRED,), jnp.int32)
    s_idx = jnp.concatenate([edge_index[0], epad])
    o_idx = jnp.concatenate([edge_index[1], epad])
    sg = jnp.concatenate([edge_index[0], zpad])
    og = jnp.concatenate([edge_index[1], zpad])
    sidx_g = sg.reshape(NW, G_NCH, G_CH)
    oidx_g = og.reshape(NW, G_NCH, G_CH)
    sidx_s = s_idx.reshape(NS, S_NCH, S_CH)
    oidx_s = o_idx.reshape(NS, S_NCH, S_CH)
    sidx_c = s_idx.reshape(NS, C_NCH, C_CH)
    oidx_c = o_idx.reshape(NS, C_NCH, C_CH)

    cnt0, cnt1 = _counts(sidx_c, oidx_c)

    ov = obj_vecs
    pv = jnp.pad(pred_vecs, ((0, E_PAD - N_PRED), (0, 0)))
    for p in params:
        w1, b1, w2, b2, w3, b3, w4, b4 = p
        b1 = b1.reshape(1, -1)
        b2 = b2.reshape(1, -1)
        b3 = b3.reshape(1, -1)
        b4 = b4.reshape(1, -1)
        ovg = ov
        if ovg.shape[1] < DOUT:
            ovg = jnp.pad(ovg, ((0, 0), (0, DOUT - ovg.shape[1])))
        gs, go = _gather(ovg, sidx_g, oidx_g)
        ns, np_, no = _edge_mlp(gs, pv, go, w1, b1, w2, b2)
        pooled = _scatter(ns, no, sidx_s, oidx_s)
        ov = _node_mlp(pooled, cnt0, cnt1, w3, b3, w4, b4)
        pv = np_
    return ov, pv[:N_PRED]


# speedup vs baseline: 1.7682x; 1.4828x over previous
import functools

import jax
import jax.numpy as jnp
from jax import lax
from jax.experimental import pallas as pl
from jax.experimental.pallas import tpu as pltpu
from jax.experimental.pallas import tpu_sc as plsc

N_OBJ = 10000
N_PRED = 160000
H = 512
DOUT = 128

NC = 2
NS = 16
NW = NC * NS

E_PAD = 163840
A_ROWS = 10008

G_EPT = E_PAD // NW
G_CH = 40
G_NCH = G_EPT // G_CH

S_EPT = E_PAD // NS
S_CH = 40
S_NCH = S_EPT // S_CH
S_NPH = 4
S_PCH = S_NCH // S_NPH
S_NSL = 4
SLAB = 128
RPT = 624
RPT_LAST = N_OBJ - (NS - 1) * RPT


def _sc_mesh():
    return plsc.VectorSubcoreMesh(core_axis_name="c", subcore_axis_name="s")


G_NSL = 4


def _gather_body(d, table, sidx, oidx, gs_out, go_out, tbl, idx_v, stage,
                 gsems, osems):
    c = lax.axis_index("c")
    s = lax.axis_index("s")
    wid = s * NC + c
    base = wid * G_EPT

    _own_rows(s, lambda r0, n: pltpu.sync_copy(
        table.at[pl.ds(r0, n)], tbl.at[pl.ds(r0, n)]))
    plsc.subcore_barrier()

    def run(idx_hbm, out_hbm):
        pltpu.sync_copy(idx_hbm.at[wid], idx_v)

        def owait(q):
            pltpu.make_async_copy(
                stage.at[q], out_hbm.at[pl.ds(base, G_CH)],
                osems.at[q]).wait()

        def flush(u, q):
            pltpu.make_async_copy(
                tbl.at[idx_v.at[0]], stage.at[q], gsems.at[q]).wait()
            pltpu.async_copy(
                stage.at[q], out_hbm.at[pl.ds(base + u * G_CH, G_CH)],
                osems.at[q])

        def quad(dd, _):
            for q in range(G_NSL):
                v = dd * G_NSL + q

                @pl.when(dd >= 1)
                def _():
                    owait(q)

                pltpu.async_copy(tbl.at[idx_v.at[v]], stage.at[q],
                                 gsems.at[q])
                qq = (q + 1) % G_NSL
                if q == G_NSL - 1:
                    flush(v - (G_NSL - 1), qq)
                else:
                    @pl.when(dd >= 1)
                    def _():
                        flush(v - (G_NSL - 1), qq)
            return ()

        lax.fori_loop(0, G_NCH // G_NSL, quad, (), unroll=False)
        for u in range(G_NCH - (G_NSL - 1), G_NCH):
            flush(u, u % G_NSL)
        for q in range(G_NSL):
            owait(q)

    run(sidx, gs_out)
    run(oidx, go_out)


def _gather(table, sidx_g, oidx_g):
    d = table.shape[1]
    kfn = pl.kernel(
        functools.partial(_gather_body, d),
        out_type=(
            jax.ShapeDtypeStruct((E_PAD, d), jnp.float32),
            jax.ShapeDtypeStruct((E_PAD, d), jnp.float32),
        ),
        mesh=_sc_mesh(),
        scratch_types=[
            pltpu.VMEM_SHARED((N_OBJ, DOUT), jnp.float32),
            pltpu.VMEM((G_NCH, G_CH), jnp.int32),
            pltpu.VMEM((G_NSL, G_CH, d), jnp.float32),
            pltpu.SemaphoreType.DMA((G_NSL,)),
            pltpu.SemaphoreType.DMA((G_NSL,)),
        ],
    )
    return kfn(table, sidx_g, oidx_g)


def _fill(ref, rows, cols, value):
    v = jnp.full((16,), value, jnp.float32)

    def zrow(r, _):
        def zcol(k, _):
            ref[r, pl.ds(k * 16, 16)] = v
            return ()
        lax.fori_loop(0, cols // 16, zcol, (), unroll=True)
        return ()

    lax.fori_loop(0, rows, zrow, (), unroll=False)


def _own_rows(s, fn):
    @pl.when(s < NS - 1)
    def _():
        fn(s * RPT, RPT)

    @pl.when(s == NS - 1)
    def _():
        fn((NS - 1) * RPT, RPT_LAST)


def _zero_rows(acc, zbuf, s):
    def do(r0, n):
        def st(i, _):
            pltpu.sync_copy(zbuf, acc.at[pl.ds(r0 + i * 8, 8)])
            return ()
        lax.fori_loop(0, n // 8, st, (), unroll=False)
    _own_rows(s, do)


def _scatter_body(vs, vo, sidx, oidx, out, acc, idx_v, vals, zbuf,
                  rsems, ssems):
    c = lax.axis_index("c")
    s = lax.axis_index("s")
    _fill(zbuf, 8, SLAB, 0.0)

    for p in range(2):
        col0 = c * (2 * SLAB) + p * SLAB
        _zero_rows(acc, zbuf, s)
        plsc.subcore_barrier()
        for idx_hbm, val_hbm in ((sidx, vs), (oidx, vo)):
            for ph in range(S_NPH):
                pltpu.sync_copy(idx_hbm.at[s, pl.ds(ph * S_PCH, S_PCH)],
                                idx_v)
                ebase = s * S_EPT + ph * S_PCH * S_CH

                def swait(q):
                    pltpu.make_async_copy(
                        vals.at[q], acc.at[pl.ds(0, S_CH)],
                        ssems.at[q]).wait()

                def add(u, q):
                    pltpu.make_async_copy(
                        val_hbm.at[pl.ds(ebase, S_CH), pl.ds(col0, SLAB)],
                        vals.at[q], rsems.at[q]).wait()
                    pltpu.async_copy(vals.at[q], acc.at[idx_v.at[u]],
                                     ssems.at[q], add=True)

                def quad(dd, _):
                    for q in range(S_NSL):
                        v = dd * S_NSL + q

                        @pl.when(dd >= 1)
                        def _():
                            swait(q)

                        pltpu.async_copy(
                            val_hbm.at[pl.ds(ebase + v * S_CH, S_CH),
                                       pl.ds(col0, SLAB)],
                            vals.at[q], rsems.at[q])
                        qq = (q + 1) % S_NSL
                        if q == S_NSL - 1:
                            add(v - (S_NSL - 1), qq)
                        else:
                            @pl.when(dd >= 1)
                            def _():
                                add(v - (S_NSL - 1), qq)
                    return ()

                lax.fori_loop(0, S_PCH // S_NSL, quad, (), unroll=False)
                for u in range(S_PCH - (S_NSL - 1), S_PCH):
                    add(u, u % S_NSL)
                for q in range(S_NSL):
                    swait(q)
        plsc.subcore_barrier()
        _own_rows(s, lambda r0, n: pltpu.sync_copy(
            acc.at[pl.ds(r0, n)],
            out.at[pl.ds(r0, n), pl.ds(col0, SLAB)]))


def _scatter(vs, vo, sidx_s, oidx_s):
    kfn = pl.kernel(
        _scatter_body,
        out_type=jax.ShapeDtypeStruct((N_OBJ, H), jnp.float32),
        mesh=_sc_mesh(),
        scratch_types=[
            pltpu.VMEM_SHARED((A_ROWS, SLAB), jnp.float32),
            pltpu.VMEM((S_PCH, S_CH), jnp.int32),
            pltpu.VMEM((S_NSL, S_CH, SLAB), jnp.float32),
            pltpu.VMEM((8, SLAB), jnp.float32),
            pltpu.SemaphoreType.DMA((S_NSL,)),
            pltpu.SemaphoreType.DMA((S_NSL,)),
        ],
    )
    return kfn(vs, vo, sidx_s, oidx_s)


CW = 128
C_CH = 64
C_NCH = S_EPT // C_CH


def _counts_body(sidx, oidx, out0, out1, acc, idx_v, ones_v, zeros_v, sem):
    c = lax.axis_index("c")
    s = lax.axis_index("s")
    _fill(ones_v, C_CH, CW, 1.0)
    _fill(zeros_v, 8, CW, 0.0)
    _zero_rows(acc, zeros_v, s)
    plsc.subcore_barrier()
    for cc, idx_hbm in ((0, sidx), (1, oidx)):
        @pl.when(c == cc)
        def _():
            pltpu.sync_copy(idx_hbm.at[s], idx_v)

            def fire(j, _):
                pltpu.async_copy(ones_v, acc.at[idx_v.at[j]], sem, add=True)
                return ()

            lax.fori_loop(0, C_NCH, fire, (), unroll=False)

            def drain(j, _):
                pltpu.make_async_copy(ones_v, acc.at[pl.ds(0, C_CH)],
                                      sem).wait()
                return ()

            lax.fori_loop(0, C_NCH, drain, (), unroll=False)
    plsc.subcore_barrier()
    for cc, out in ((0, out0), (1, out1)):
        @pl.when(c == cc)
        def _():
            _own_rows(s, lambda r0, n: pltpu.sync_copy(
                acc.at[pl.ds(r0, n)], out.at[pl.ds(r0, n)]))


def _counts(sidx_c, oidx_c):
    kfn = pl.kernel(
        _counts_body,
        out_type=(
            jax.ShapeDtypeStruct((N_OBJ, CW), jnp.float32),
            jax.ShapeDtypeStruct((N_OBJ, CW), jnp.float32),
        ),
        mesh=_sc_mesh(),
        scratch_types=[
            pltpu.VMEM_SHARED((A_ROWS, CW), jnp.float32),
            pltpu.VMEM((C_NCH, C_CH), jnp.int32),
            pltpu.VMEM((C_CH, CW), jnp.float32),
            pltpu.VMEM((8, CW), jnp.float32),
            pltpu.SemaphoreType.DMA,
        ],
    )
    return kfn(sidx_c, oidx_c)


BE = 1024
BN = 1000


def _edge_mlp_body(din, gs, pred, go, w1, b1, w2, b2, ns, np_, no):
    h = jnp.dot(gs[:, :din], w1[:din, :], preferred_element_type=jnp.float32)
    h += jnp.dot(pred[...], w1[din:2 * din, :],
                 preferred_element_type=jnp.float32)
    h += jnp.dot(go[:, :din], w1[2 * din:, :],
                 preferred_element_type=jnp.float32)
    h = jax.nn.relu(h + b1[...])
    ns[...] = jax.nn.relu(
        jnp.dot(h, w2[:, :H], preferred_element_type=jnp.float32)
        + b2[:, :H])
    np_[...] = jax.nn.relu(
        jnp.dot(h, w2[:, H:H + DOUT], preferred_element_type=jnp.float32)
        + b2[:, H:H + DOUT])
    no[...] = jax.nn.relu(
        jnp.dot(h, w2[:, H + DOUT:], preferred_element_type=jnp.float32)
        + b2[:, H + DOUT:])


def _edge_mlp(gs, pred, go, w1, b1, w2, b2):
    din = w1.shape[0] // 3
    dg = gs.shape[1]
    grid = (E_PAD // BE,)
    row = lambda i: (i, 0)
    full = lambda i: (0, 0)
    return pl.pallas_call(
        functools.partial(_edge_mlp_body, din),
        grid=grid,
        in_specs=[
            pl.BlockSpec((BE, dg), row),
            pl.BlockSpec((BE, din), row),
            pl.BlockSpec((BE, dg), row),
            pl.BlockSpec(w1.shape, full),
            pl.BlockSpec(b1.shape, full),
            pl.BlockSpec(w2.shape, full),
            pl.BlockSpec(b2.shape, full),
        ],
        out_specs=[
            pl.BlockSpec((BE, H), row),
            pl.BlockSpec((BE, DOUT), row),
            pl.BlockSpec((BE, H), row),
        ],
        out_shape=[
            jax.ShapeDtypeStruct((E_PAD, H), jnp.float32),
            jax.ShapeDtypeStruct((E_PAD, DOUT), jnp.float32),
            jax.ShapeDtypeStruct((E_PAD, H), jnp.float32),
        ],
    )(gs, pred, go, w1, b1, w2, b2)


def _node_mlp_body(pooled, cnt0, cnt1, w3, b3, w4, b4, out):
    c = cnt0[:, 0:1] + cnt1[:, 0:1]
    inv = 1.0 / jnp.maximum(c, 1.0)
    h2 = jax.nn.relu(
        jnp.dot(pooled[...] * inv, w3[...], preferred_element_type=jnp.float32)
        + b3[...])
    out[...] = jnp.dot(h2, w4[...], preferred_element_type=jnp.float32) + b4[...]


def _node_mlp(pooled, cnt0, cnt1, w3, b3, w4, b4):
    grid = (N_OBJ // BN,)
    row = lambda i: (i, 0)
    full = lambda i: (0, 0)
    return pl.pallas_call(
        _node_mlp_body,
        grid=grid,
        in_specs=[
            pl.BlockSpec((BN, H), row),
            pl.BlockSpec((BN, CW), row),
            pl.BlockSpec((BN, CW), row),
            pl.BlockSpec(w3.shape, full),
            pl.BlockSpec(b3.shape, full),
            pl.BlockSpec(w4.shape, full),
            pl.BlockSpec(b4.shape, full),
        ],
        out_specs=pl.BlockSpec((BN, DOUT), row),
        out_shape=jax.ShapeDtypeStruct((N_OBJ, DOUT), jnp.float32),
    )(pooled, cnt0, cnt1, w3, b3, w4, b4)


def kernel(obj_vecs, pred_vecs, edge_index, params):
    epad = jnp.full((E_PAD - N_PRED,), N_OBJ, jnp.int32)
    zpad = jnp.zeros((E_PAD - N_PRED,), jnp.int32)
    s_idx = jnp.concatenate([edge_index[0], epad])
    o_idx = jnp.concatenate([edge_index[1], epad])
    sg = jnp.concatenate([edge_index[0], zpad])
    og = jnp.concatenate([edge_index[1], zpad])
    sidx_g = sg.reshape(NW, G_NCH, G_CH)
    oidx_g = og.reshape(NW, G_NCH, G_CH)
    sidx_s = s_idx.reshape(NS, S_NCH, S_CH)
    oidx_s = o_idx.reshape(NS, S_NCH, S_CH)
    sidx_c = s_idx.reshape(NS, C_NCH, C_CH)
    oidx_c = o_idx.reshape(NS, C_NCH, C_CH)

    cnt0, cnt1 = _counts(sidx_c, oidx_c)

    ov = obj_vecs
    pv = jnp.pad(pred_vecs, ((0, E_PAD - N_PRED), (0, 0)))
    for p in params:
        w1, b1, w2, b2, w3, b3, w4, b4 = p
        b1 = b1.reshape(1, -1)
        b2 = b2.reshape(1, -1)
        b3 = b3.reshape(1, -1)
        b4 = b4.reshape(1, -1)
        ovg = ov
        if ovg.shape[1] < DOUT:
            ovg = jnp.pad(ovg, ((0, 0), (0, DOUT - ovg.shape[1])))
        gs, go = _gather(ovg, sidx_g, oidx_g)
        ns, np_, no = _edge_mlp(gs, pv, go, w1, b1, w2, b2)
        pooled = _scatter(ns, no, sidx_s, oidx_s)
        ov = _node_mlp(pooled, cnt0, cnt1, w3, b3, w4, b4)
        pv = np_
    return ov, pv[:N_PRED]


# speedup vs baseline: 1.7687x; 1.0003x over previous
import functools

import jax
import jax.numpy as jnp
from jax import lax
from jax.experimental import pallas as pl
from jax.experimental.pallas import tpu as pltpu
from jax.experimental.pallas import tpu_sc as plsc

N_OBJ = 10000
N_PRED = 160000
H = 512
DOUT = 128

NC = 2
NS = 16
NW = NC * NS

E_PAD = 163840
A_ROWS = 10008

G_EPT = E_PAD // NW
G_CH = 40
G_NCH = G_EPT // G_CH

S_EPT = E_PAD // NS
S_CH = 40
S_NCH = S_EPT // S_CH
S_NPH = 4
S_PCH = S_NCH // S_NPH
S_NSL = 4
SLAB = 128
RPT = 624
RPT_LAST = N_OBJ - (NS - 1) * RPT


def _sc_mesh():
    return plsc.VectorSubcoreMesh(core_axis_name="c", subcore_axis_name="s")


G_NSL = 4


def _gather_body(d, table, sidx, oidx, gs_out, go_out, tbl, idx_v, stage,
                 gsems, osems):
    c = lax.axis_index("c")
    s = lax.axis_index("s")
    wid = s * NC + c
    base = wid * G_EPT

    _own_rows(s, lambda r0, n: pltpu.sync_copy(
        table.at[pl.ds(r0, n)], tbl.at[pl.ds(r0, n)]))
    plsc.subcore_barrier()

    def run(idx_hbm, out_hbm):
        pltpu.sync_copy(idx_hbm.at[wid], idx_v)

        def owait(q):
            pltpu.make_async_copy(
                stage.at[q], out_hbm.at[pl.ds(base, G_CH)],
                osems.at[q]).wait()

        def flush(u, q):
            pltpu.make_async_copy(
                tbl.at[idx_v.at[0]], stage.at[q], gsems.at[q]).wait()
            pltpu.async_copy(
                stage.at[q], out_hbm.at[pl.ds(base + u * G_CH, G_CH)],
                osems.at[q])

        def quad(dd, _):
            for q in range(G_NSL):
                v = dd * G_NSL + q

                @pl.when(dd >= 1)
                def _():
                    owait(q)

                pltpu.async_copy(tbl.at[idx_v.at[v]], stage.at[q],
                                 gsems.at[q])
                qq = (q + 1) % G_NSL
                if q == G_NSL - 1:
                    flush(v - (G_NSL - 1), qq)
                else:
                    @pl.when(dd >= 1)
                    def _():
                        flush(v - (G_NSL - 1), qq)
            return ()

        lax.fori_loop(0, G_NCH // G_NSL, quad, (), unroll=False)
        for u in range(G_NCH - (G_NSL - 1), G_NCH):
            flush(u, u % G_NSL)
        for q in range(G_NSL):
            owait(q)

    run(sidx, gs_out)
    run(oidx, go_out)


def _gather(table, sidx_g, oidx_g):
    d = table.shape[1]
    kfn = pl.kernel(
        functools.partial(_gather_body, d),
        out_type=(
            jax.ShapeDtypeStruct((E_PAD, d), jnp.float32),
            jax.ShapeDtypeStruct((E_PAD, d), jnp.float32),
        ),
        mesh=_sc_mesh(),
        scratch_types=[
            pltpu.VMEM_SHARED((N_OBJ, DOUT), jnp.float32),
            pltpu.VMEM((G_NCH, G_CH), jnp.int32),
            pltpu.VMEM((G_NSL, G_CH, d), jnp.float32),
            pltpu.SemaphoreType.DMA((G_NSL,)),
            pltpu.SemaphoreType.DMA((G_NSL,)),
        ],
    )
    return kfn(table, sidx_g, oidx_g)


def _fill(ref, rows, cols, value):
    v = jnp.full((16,), value, jnp.float32)

    def zrow(r, _):
        def zcol(k, _):
            ref[r, pl.ds(k * 16, 16)] = v
            return ()
        lax.fori_loop(0, cols // 16, zcol, (), unroll=True)
        return ()

    lax.fori_loop(0, rows, zrow, (), unroll=False)


def _own_rows(s, fn):
    @pl.when(s < NS - 1)
    def _():
        fn(s * RPT, RPT)

    @pl.when(s == NS - 1)
    def _():
        fn((NS - 1) * RPT, RPT_LAST)


def _zero_rows(acc, zbuf, s):
    def do(r0, n):
        def st(i, _):
            pltpu.sync_copy(zbuf, acc.at[pl.ds(r0 + i * 8, 8)])
            return ()
        lax.fori_loop(0, n // 8, st, (), unroll=False)
    _own_rows(s, do)


def _scatter_body(vs, vo, sidx, oidx, out, acc, idx_v, vals, zbuf,
                  rsems, ssems):
    c = lax.axis_index("c")
    s = lax.axis_index("s")
    _fill(zbuf, 8, SLAB, 0.0)

    for p in range(2):
        col0 = c * (2 * SLAB) + p * SLAB
        _zero_rows(acc, zbuf, s)
        plsc.subcore_barrier()
        for idx_hbm, val_hbm in ((sidx, vs), (oidx, vo)):
            for ph in range(S_NPH):
                pltpu.sync_copy(idx_hbm.at[s, pl.ds(ph * S_PCH, S_PCH)],
                                idx_v)
                ebase = s * S_EPT + ph * S_PCH * S_CH

                def swait(q):
                    pltpu.make_async_copy(
                        vals.at[q], acc.at[pl.ds(0, S_CH)],
                        ssems.at[q]).wait()

                def add(u, q):
                    pltpu.make_async_copy(
                        val_hbm.at[pl.ds(ebase, S_CH), pl.ds(col0, SLAB)],
                        vals.at[q], rsems.at[q]).wait()
                    pltpu.async_copy(vals.at[q], acc.at[idx_v.at[u]],
                                     ssems.at[q], add=True)

                def quad(dd, _):
                    for q in range(S_NSL):
                        v = dd * S_NSL + q

                        @pl.when(dd >= 1)
                        def _():
                            swait(q)

                        pltpu.async_copy(
                            val_hbm.at[pl.ds(ebase + v * S_CH, S_CH),
                                       pl.ds(col0, SLAB)],
                            vals.at[q], rsems.at[q])
                        qq = (q + 1) % S_NSL
                        if q == S_NSL - 1:
                            add(v - (S_NSL - 1), qq)
                        else:
                            @pl.when(dd >= 1)
                            def _():
                                add(v - (S_NSL - 1), qq)
                    return ()

                lax.fori_loop(0, S_PCH // S_NSL, quad, (), unroll=False)
                for u in range(S_PCH - (S_NSL - 1), S_PCH):
                    add(u, u % S_NSL)
                for q in range(S_NSL):
                    swait(q)
        plsc.subcore_barrier()
        _own_rows(s, lambda r0, n: pltpu.sync_copy(
            acc.at[pl.ds(r0, n)],
            out.at[pl.ds(r0, n), pl.ds(col0, SLAB)]))


def _scatter(vs, vo, sidx_s, oidx_s):
    kfn = pl.kernel(
        _scatter_body,
        out_type=jax.ShapeDtypeStruct((N_OBJ, H), jnp.float32),
        mesh=_sc_mesh(),
        scratch_types=[
            pltpu.VMEM_SHARED((A_ROWS, SLAB), jnp.float32),
            pltpu.VMEM((S_PCH, S_CH), jnp.int32),
            pltpu.VMEM((S_NSL, S_CH, SLAB), jnp.float32),
            pltpu.VMEM((8, SLAB), jnp.float32),
            pltpu.SemaphoreType.DMA((S_NSL,)),
            pltpu.SemaphoreType.DMA((S_NSL,)),
        ],
    )
    return kfn(vs, vo, sidx_s, oidx_s)


CW = 128
C_CH = 64
C_NCH = S_EPT // C_CH


def _counts_body(sidx, oidx, out0, out1, acc, idx_v, ones_v, zeros_v, sem):
    c = lax.axis_index("c")
    s = lax.axis_index("s")
    _fill(ones_v, C_CH, CW, 1.0)
    _fill(zeros_v, 8, CW, 0.0)
    _zero_rows(acc, zeros_v, s)
    plsc.subcore_barrier()
    for cc, idx_hbm in ((0, sidx), (1, oidx)):
        @pl.when(c == cc)
        def _():
            pltpu.sync_copy(idx_hbm.at[s], idx_v)

            def fire(j, _):
                pltpu.async_copy(ones_v, acc.at[idx_v.at[j]], sem, add=True)
                return ()

            lax.fori_loop(0, C_NCH, fire, (), unroll=False)

            def drain(j, _):
                pltpu.make_async_copy(ones_v, acc.at[pl.ds(0, C_CH)],
                                      sem).wait()
                return ()

            lax.fori_loop(0, C_NCH, drain, (), unroll=False)
    plsc.subcore_barrier()
    for cc, out in ((0, out0), (1, out1)):
        @pl.when(c == cc)
        def _():
            _own_rows(s, lambda r0, n: pltpu.sync_copy(
                acc.at[pl.ds(r0, n)], out.at[pl.ds(r0, n)]))


def _counts(sidx_c, oidx_c):
    kfn = pl.kernel(
        _counts_body,
        out_type=(
            jax.ShapeDtypeStruct((N_OBJ, CW), jnp.float32),
            jax.ShapeDtypeStruct((N_OBJ, CW), jnp.float32),
        ),
        mesh=_sc_mesh(),
        scratch_types=[
            pltpu.VMEM_SHARED((A_ROWS, CW), jnp.float32),
            pltpu.VMEM((C_NCH, C_CH), jnp.int32),
            pltpu.VMEM((C_CH, CW), jnp.float32),
            pltpu.VMEM((8, CW), jnp.float32),
            pltpu.SemaphoreType.DMA,
        ],
    )
    return kfn(sidx_c, oidx_c)


BE = 1024
BN = 1000


def _edge_mlp_body(din, gs, pred, go, w1, b1, w2, b2, ns, np_, no):
    h = jnp.dot(gs[:, :din], w1[:din, :], preferred_element_type=jnp.float32)
    h += jnp.dot(pred[...], w1[din:2 * din, :],
                 preferred_element_type=jnp.float32)
    h += jnp.dot(go[:, :din], w1[2 * din:, :],
                 preferred_element_type=jnp.float32)
    h = jax.nn.relu(h + b1[...]).astype(jnp.bfloat16)
    ns[...] = jax.nn.relu(
        jnp.dot(h, w2[:, :H], preferred_element_type=jnp.float32)
        + b2[:, :H])
    np_[...] = jax.nn.relu(
        jnp.dot(h, w2[:, H:H + DOUT], preferred_element_type=jnp.float32)
        + b2[:, H:H + DOUT])
    no[...] = jax.nn.relu(
        jnp.dot(h, w2[:, H + DOUT:], preferred_element_type=jnp.float32)
        + b2[:, H + DOUT:])


def _edge_mlp(gs, pred, go, w1, b1, w2, b2):
    din = w1.shape[0] // 3
    dg = gs.shape[1]
    grid = (E_PAD // BE,)
    row = lambda i: (i, 0)
    full = lambda i: (0, 0)
    return pl.pallas_call(
        functools.partial(_edge_mlp_body, din),
        grid=grid,
        in_specs=[
            pl.BlockSpec((BE, dg), row),
            pl.BlockSpec((BE, din), row),
            pl.BlockSpec((BE, dg), row),
            pl.BlockSpec(w1.shape, full),
            pl.BlockSpec(b1.shape, full),
            pl.BlockSpec(w2.shape, full),
            pl.BlockSpec(b2.shape, full),
        ],
        out_specs=[
            pl.BlockSpec((BE, H), row),
            pl.BlockSpec((BE, DOUT), row),
            pl.BlockSpec((BE, H), row),
        ],
        out_shape=[
            jax.ShapeDtypeStruct((E_PAD, H), jnp.float32),
            jax.ShapeDtypeStruct((E_PAD, DOUT), jnp.float32),
            jax.ShapeDtypeStruct((E_PAD, H), jnp.float32),
        ],
    )(gs, pred, go, w1, b1, w2, b2)


def _node_mlp_body(pooled, cnt0, cnt1, w3, b3, w4, b4, out):
    c = cnt0[:, 0:1] + cnt1[:, 0:1]
    inv = 1.0 / jnp.maximum(c, 1.0)
    h2 = jax.nn.relu(
        jnp.dot(pooled[...] * inv, w3[...], preferred_element_type=jnp.float32)
        + b3[...])
    out[...] = jnp.dot(h2, w4[...], preferred_element_type=jnp.float32) + b4[...]


def _node_mlp(pooled, cnt0, cnt1, w3, b3, w4, b4):
    grid = (N_OBJ // BN,)
    row = lambda i: (i, 0)
    full = lambda i: (0, 0)
    return pl.pallas_call(
        _node_mlp_body,
        grid=grid,
        in_specs=[
            pl.BlockSpec((BN, H), row),
            pl.BlockSpec((BN, CW), row),
            pl.BlockSpec((BN, CW), row),
            pl.BlockSpec(w3.shape, full),
            pl.BlockSpec(b3.shape, full),
            pl.BlockSpec(w4.shape, full),
            pl.BlockSpec(b4.shape, full),
        ],
        out_specs=pl.BlockSpec((BN, DOUT), row),
        out_shape=jax.ShapeDtypeStruct((N_OBJ, DOUT), jnp.float32),
    )(pooled, cnt0, cnt1, w3, b3, w4, b4)


def kernel(obj_vecs, pred_vecs, edge_index, params):
    epad = jnp.full((E_PAD - N_PRED,), N_OBJ, jnp.int32)
    zpad = jnp.zeros((E_PAD - N_PRED,), jnp.int32)
    s_idx = jnp.concatenate([edge_index[0], epad])
    o_idx = jnp.concatenate([edge_index[1], epad])
    sg = jnp.concatenate([edge_index[0], zpad])
    og = jnp.concatenate([edge_index[1], zpad])
    sidx_g = sg.reshape(NW, G_NCH, G_CH)
    oidx_g = og.reshape(NW, G_NCH, G_CH)
    sidx_s = s_idx.reshape(NS, S_NCH, S_CH)
    oidx_s = o_idx.reshape(NS, S_NCH, S_CH)
    sidx_c = s_idx.reshape(NS, C_NCH, C_CH)
    oidx_c = o_idx.reshape(NS, C_NCH, C_CH)

    cnt0, cnt1 = _counts(sidx_c, oidx_c)

    ov = obj_vecs
    pv = jnp.pad(pred_vecs, ((0, E_PAD - N_PRED), (0, 0)))
    for p in params:
        w1, b1, w2, b2, w3, b3, w4, b4 = p
        b1 = b1.reshape(1, -1)
        b2 = b2.reshape(1, -1)
        b3 = b3.reshape(1, -1)
        b4 = b4.reshape(1, -1)
        ovg = ov
        if ovg.shape[1] < DOUT:
            ovg = jnp.pad(ovg, ((0, 0), (0, DOUT - ovg.shape[1])))
        gs, go = _gather(ovg, sidx_g, oidx_g)
        ns, np_, no = _edge_mlp(gs, pv, go, w1, b1,
                                w2.astype(jnp.bfloat16), b2)
        pooled = _scatter(ns, no, sidx_s, oidx_s)
        ov = _node_mlp(pooled, cnt0, cnt1, w3, b3, w4, b4)
        pv = np_
    return ov, pv[:N_PRED]


# speedup vs baseline: 1.8859x; 1.0662x over previous
import functools

import jax
import jax.numpy as jnp
from jax import lax
from jax.experimental import pallas as pl
from jax.experimental.pallas import tpu as pltpu
from jax.experimental.pallas import tpu_sc as plsc

N_OBJ = 10000
N_PRED = 160000
H = 512
DOUT = 128

NC = 2
NS = 16
NW = NC * NS

E_PAD = 163840
A_ROWS = 10008

G_EPT = E_PAD // NW
G_CH = 40
G_NCH = G_EPT // G_CH

NHALF = 2
E_HALF = E_PAD // NHALF
S_EPT = E_HALF // NS
S_CH = 40
S_NCH = S_EPT // S_CH
S_PCH = 64
S_NPH = S_NCH // S_PCH
S_NSL = 4
SLAB = 128
RPT = 624
RPT_LAST = N_OBJ - (NS - 1) * RPT


def _sc_mesh():
    return plsc.VectorSubcoreMesh(core_axis_name="c", subcore_axis_name="s")


G_NSL = 4


def _gather_body(d, table, sidx, oidx, gs_out, go_out, tbl, idx_v, stage,
                 gsems, osems):
    c = lax.axis_index("c")
    s = lax.axis_index("s")
    wid = s * NC + c
    base = wid * G_EPT

    _own_rows(s, lambda r0, n: pltpu.sync_copy(
        table.at[pl.ds(r0, n)], tbl.at[pl.ds(r0, n)]))
    plsc.subcore_barrier()

    def run(idx_hbm, out_hbm):
        pltpu.sync_copy(idx_hbm.at[wid], idx_v)

        def owait(q):
            pltpu.make_async_copy(
                stage.at[q], out_hbm.at[pl.ds(base, G_CH)],
                osems.at[q]).wait()

        def flush(u, q):
            pltpu.make_async_copy(
                tbl.at[idx_v.at[0]], stage.at[q], gsems.at[q]).wait()
            pltpu.async_copy(
                stage.at[q], out_hbm.at[pl.ds(base + u * G_CH, G_CH)],
                osems.at[q])

        def quad(dd, _):
            for q in range(G_NSL):
                v = dd * G_NSL + q

                @pl.when(dd >= 1)
                def _():
                    owait(q)

                pltpu.async_copy(tbl.at[idx_v.at[v]], stage.at[q],
                                 gsems.at[q])
                qq = (q + 1) % G_NSL
                if q == G_NSL - 1:
                    flush(v - (G_NSL - 1), qq)
                else:
                    @pl.when(dd >= 1)
                    def _():
                        flush(v - (G_NSL - 1), qq)
            return ()

        lax.fori_loop(0, G_NCH // G_NSL, quad, (), unroll=False)
        for u in range(G_NCH - (G_NSL - 1), G_NCH):
            flush(u, u % G_NSL)
        for q in range(G_NSL):
            owait(q)

    run(sidx, gs_out)
    run(oidx, go_out)


def _gather(table, sidx_g, oidx_g):
    d = table.shape[1]
    kfn = pl.kernel(
        functools.partial(_gather_body, d),
        out_type=(
            jax.ShapeDtypeStruct((E_PAD, d), jnp.float32),
            jax.ShapeDtypeStruct((E_PAD, d), jnp.float32),
        ),
        mesh=_sc_mesh(),
        scratch_types=[
            pltpu.VMEM_SHARED((N_OBJ, DOUT), jnp.float32),
            pltpu.VMEM((G_NCH, G_CH), jnp.int32),
            pltpu.VMEM((G_NSL, G_CH, d), jnp.float32),
            pltpu.SemaphoreType.DMA((G_NSL,)),
            pltpu.SemaphoreType.DMA((G_NSL,)),
        ],
    )
    return kfn(table, sidx_g, oidx_g)


def _fill(ref, rows, cols, value):
    v = jnp.full((16,), value, jnp.float32)

    def zrow(r, _):
        def zcol(k, _):
            ref[r, pl.ds(k * 16, 16)] = v
            return ()
        lax.fori_loop(0, cols // 16, zcol, (), unroll=True)
        return ()

    lax.fori_loop(0, rows, zrow, (), unroll=False)


def _own_rows(s, fn):
    @pl.when(s < NS - 1)
    def _():
        fn(s * RPT, RPT)

    @pl.when(s == NS - 1)
    def _():
        fn((NS - 1) * RPT, RPT_LAST)


def _zero_rows(acc, zbuf, s):
    def do(r0, n):
        def st(i, _):
            pltpu.sync_copy(zbuf, acc.at[pl.ds(r0 + i * 8, 8)])
            return ()
        lax.fori_loop(0, n // 8, st, (), unroll=False)
    _own_rows(s, do)


def _scatter_body(vs, vo, sidx, oidx, out, acc, idx_v, vals, zbuf,
                  rsems, ssems):
    c = lax.axis_index("c")
    s = lax.axis_index("s")
    _fill(zbuf, 8, SLAB, 0.0)

    for p in range(2):
        col0 = c * (2 * SLAB) + p * SLAB
        _zero_rows(acc, zbuf, s)
        plsc.subcore_barrier()
        for idx_hbm, val_hbm in ((sidx, vs), (oidx, vo)):
            for ph in range(S_NPH):
                pltpu.sync_copy(idx_hbm.at[s, pl.ds(ph * S_PCH, S_PCH)],
                                idx_v)
                ebase = s * S_EPT + ph * S_PCH * S_CH

                def swait(q):
                    pltpu.make_async_copy(
                        vals.at[q], acc.at[pl.ds(0, S_CH)],
                        ssems.at[q]).wait()

                def add(u, q):
                    pltpu.make_async_copy(
                        val_hbm.at[pl.ds(ebase, S_CH), pl.ds(col0, SLAB)],
                        vals.at[q], rsems.at[q]).wait()
                    pltpu.async_copy(vals.at[q], acc.at[idx_v.at[u]],
                                     ssems.at[q], add=True)

                def quad(dd, _):
                    for q in range(S_NSL):
                        v = dd * S_NSL + q

                        @pl.when(dd >= 1)
                        def _():
                            swait(q)

                        pltpu.async_copy(
                            val_hbm.at[pl.ds(ebase + v * S_CH, S_CH),
                                       pl.ds(col0, SLAB)],
                            vals.at[q], rsems.at[q])
                        qq = (q + 1) % S_NSL
                        if q == S_NSL - 1:
                            add(v - (S_NSL - 1), qq)
                        else:
                            @pl.when(dd >= 1)
                            def _():
                                add(v - (S_NSL - 1), qq)
                    return ()

                lax.fori_loop(0, S_PCH // S_NSL, quad, (), unroll=False)
                for u in range(S_PCH - (S_NSL - 1), S_PCH):
                    add(u, u % S_NSL)
                for q in range(S_NSL):
                    swait(q)
        plsc.subcore_barrier()
        _own_rows(s, lambda r0, n: pltpu.sync_copy(
            acc.at[pl.ds(r0, n)],
            out.at[pl.ds(r0, n), pl.ds(col0, SLAB)]))


def _scatter(vs, vo, sidx_s, oidx_s):
    kfn = pl.kernel(
        _scatter_body,
        out_type=jax.ShapeDtypeStruct((N_OBJ, H), jnp.float32),
        mesh=_sc_mesh(),
        scratch_types=[
            pltpu.VMEM_SHARED((A_ROWS, SLAB), jnp.float32),
            pltpu.VMEM((S_PCH, S_CH), jnp.int32),
            pltpu.VMEM((S_NSL, S_CH, SLAB), jnp.float32),
            pltpu.VMEM((8, SLAB), jnp.float32),
            pltpu.SemaphoreType.DMA((S_NSL,)),
            pltpu.SemaphoreType.DMA((S_NSL,)),
        ],
    )
    return kfn(vs, vo, sidx_s, oidx_s)


CW = 128
C_CH = 64
C_EPT = E_PAD // NS
C_NCH = C_EPT // C_CH


def _counts_body(sidx, oidx, out0, out1, acc, idx_v, ones_v, zeros_v, sem):
    c = lax.axis_index("c")
    s = lax.axis_index("s")
    _fill(ones_v, C_CH, CW, 1.0)
    _fill(zeros_v, 8, CW, 0.0)
    _zero_rows(acc, zeros_v, s)
    plsc.subcore_barrier()
    for cc, idx_hbm in ((0, sidx), (1, oidx)):
        @pl.when(c == cc)
        def _():
            pltpu.sync_copy(idx_hbm.at[s], idx_v)

            def fire(j, _):
                pltpu.async_copy(ones_v, acc.at[idx_v.at[j]], sem, add=True)
                return ()

            lax.fori_loop(0, C_NCH, fire, (), unroll=False)

            def drain(j, _):
                pltpu.make_async_copy(ones_v, acc.at[pl.ds(0, C_CH)],
                                      sem).wait()
                return ()

            lax.fori_loop(0, C_NCH, drain, (), unroll=False)
    plsc.subcore_barrier()
    for cc, out in ((0, out0), (1, out1)):
        @pl.when(c == cc)
        def _():
            _own_rows(s, lambda r0, n: pltpu.sync_copy(
                acc.at[pl.ds(r0, n)], out.at[pl.ds(r0, n)]))


def _counts(sidx_c, oidx_c):
    kfn = pl.kernel(
        _counts_body,
        out_type=(
            jax.ShapeDtypeStruct((N_OBJ, CW), jnp.float32),
            jax.ShapeDtypeStruct((N_OBJ, CW), jnp.float32),
        ),
        mesh=_sc_mesh(),
        scratch_types=[
            pltpu.VMEM_SHARED((A_ROWS, CW), jnp.float32),
            pltpu.VMEM((C_NCH, C_CH), jnp.int32),
            pltpu.VMEM((C_CH, CW), jnp.float32),
            pltpu.VMEM((8, CW), jnp.float32),
            pltpu.SemaphoreType.DMA,
        ],
    )
    return kfn(sidx_c, oidx_c)


BE = 1024
BN = 1000


def _edge_mlp_body(din, gs, pred, go, w1, b1, w2, b2, ns, np_, no):
    h = jnp.dot(gs[:, :din], w1[:din, :], preferred_element_type=jnp.float32)
    h += jnp.dot(pred[...], w1[din:2 * din, :],
                 preferred_element_type=jnp.float32)
    h += jnp.dot(go[:, :din], w1[2 * din:, :],
                 preferred_element_type=jnp.float32)
    h = jax.nn.relu(h + b1[...]).astype(jnp.bfloat16)
    ns[...] = jax.nn.relu(
        jnp.dot(h, w2[:, :H], preferred_element_type=jnp.float32)
        + b2[:, :H])
    np_[...] = jax.nn.relu(
        jnp.dot(h, w2[:, H:H + DOUT], preferred_element_type=jnp.float32)
        + b2[:, H:H + DOUT])
    no[...] = jax.nn.relu(
        jnp.dot(h, w2[:, H + DOUT:], preferred_element_type=jnp.float32)
        + b2[:, H + DOUT:])


def _edge_mlp(gs, pred, go, w1, b1, w2, b2, half):
    din = w1.shape[0] // 3
    dg = gs.shape[1]
    steps = E_HALF // BE
    grid = (steps,)
    row = lambda i: (i, 0)
    offrow = lambda i: (i + half * steps, 0)
    full = lambda i: (0, 0)
    return pl.pallas_call(
        functools.partial(_edge_mlp_body, din),
        grid=grid,
        in_specs=[
            pl.BlockSpec((BE, dg), offrow),
            pl.BlockSpec((BE, din), row),
            pl.BlockSpec((BE, dg), offrow),
            pl.BlockSpec(w1.shape, full),
            pl.BlockSpec(b1.shape, full),
            pl.BlockSpec(w2.shape, full),
            pl.BlockSpec(b2.shape, full),
        ],
        out_specs=[
            pl.BlockSpec((BE, H), row),
            pl.BlockSpec((BE, DOUT), row),
            pl.BlockSpec((BE, H), row),
        ],
        out_shape=[
            jax.ShapeDtypeStruct((E_HALF, H), jnp.float32),
            jax.ShapeDtypeStruct((E_HALF, DOUT), jnp.float32),
            jax.ShapeDtypeStruct((E_HALF, H), jnp.float32),
        ],
    )(gs, pred, go, w1, b1, w2, b2)


def _node_mlp_body(pooled_a, pooled_b, cnt0, cnt1, w3, b3, w4, b4, out):
    c = cnt0[:, 0:1] + cnt1[:, 0:1]
    inv = 1.0 / jnp.maximum(c, 1.0)
    pooled = pooled_a[...] + pooled_b[...]
    h2 = jax.nn.relu(
        jnp.dot(pooled * inv, w3[...], preferred_element_type=jnp.float32)
        + b3[...])
    out[...] = jnp.dot(h2, w4[...], preferred_element_type=jnp.float32) + b4[...]


def _node_mlp(pooled_a, pooled_b, cnt0, cnt1, w3, b3, w4, b4):
    grid = (N_OBJ // BN,)
    row = lambda i: (i, 0)
    full = lambda i: (0, 0)
    return pl.pallas_call(
        _node_mlp_body,
        grid=grid,
        in_specs=[
            pl.BlockSpec((BN, H), row),
            pl.BlockSpec((BN, H), row),
            pl.BlockSpec((BN, CW), row),
            pl.BlockSpec((BN, CW), row),
            pl.BlockSpec(w3.shape, full),
            pl.BlockSpec(b3.shape, full),
            pl.BlockSpec(w4.shape, full),
            pl.BlockSpec(b4.shape, full),
        ],
        out_specs=pl.BlockSpec((BN, DOUT), row),
        out_shape=jax.ShapeDtypeStruct((N_OBJ, DOUT), jnp.float32),
    )(pooled_a, pooled_b, cnt0, cnt1, w3, b3, w4, b4)


def kernel(obj_vecs, pred_vecs, edge_index, params):
    epad = jnp.full((E_PAD - N_PRED,), N_OBJ, jnp.int32)
    zpad = jnp.zeros((E_PAD - N_PRED,), jnp.int32)
    s_idx = jnp.concatenate([edge_index[0], epad])
    o_idx = jnp.concatenate([edge_index[1], epad])
    sg = jnp.concatenate([edge_index[0], zpad])
    og = jnp.concatenate([edge_index[1], zpad])
    sidx_g = sg.reshape(NW, G_NCH, G_CH)
    oidx_g = og.reshape(NW, G_NCH, G_CH)
    sidx_h = [s_idx[h * E_HALF:(h + 1) * E_HALF].reshape(NS, S_NCH, S_CH)
              for h in range(NHALF)]
    oidx_h = [o_idx[h * E_HALF:(h + 1) * E_HALF].reshape(NS, S_NCH, S_CH)
              for h in range(NHALF)]
    sidx_c = s_idx.reshape(NS, C_NCH, C_CH)
    oidx_c = o_idx.reshape(NS, C_NCH, C_CH)

    cnt0, cnt1 = _counts(sidx_c, oidx_c)

    ov = obj_vecs
    pvp = jnp.pad(pred_vecs, ((0, E_PAD - N_PRED), (0, 0)))
    pv = [pvp[h * E_HALF:(h + 1) * E_HALF] for h in range(NHALF)]
    for p in params:
        w1, b1, w2, b2, w3, b3, w4, b4 = p
        b1 = b1.reshape(1, -1)
        b2 = b2.reshape(1, -1)
        b3 = b3.reshape(1, -1)
        b4 = b4.reshape(1, -1)
        ovg = ov
        if ovg.shape[1] < DOUT:
            ovg = jnp.pad(ovg, ((0, 0), (0, DOUT - ovg.shape[1])))
        gs, go = _gather(ovg, sidx_g, oidx_g)
        w2b = w2.astype(jnp.bfloat16)
        new_pv, pooled = [], []
        for h in range(NHALF):
            ns, np_, no = _edge_mlp(gs, pv[h], go, w1, b1, w2b, b2, h)
            pooled.append(_scatter(ns, no, sidx_h[h], oidx_h[h]))
            new_pv.append(np_)
        ov = _node_mlp(pooled[0], pooled[1], cnt0, cnt1, w3, b3, w4, b4)
        pv = new_pv
    return ov, jnp.concatenate(pv)[:N_PRED]
